# Initial kernel scaffold; baseline (speedup 1.0000x reference)
import jax
import jax.numpy as jnp
from jax.experimental import pallas as pl


def kernel(x_s, x_t, edge_attr, edge_index, x_s_batch, x_t_batch, y, params):
    raise NotImplementedError("write your pallas kernel here")


# speedup vs baseline: 3.0335x; 3.0335x over previous
import functools

import jax
import jax.numpy as jnp
from jax import lax
from jax.experimental import pallas as pl
from jax.experimental.pallas import tpu as pltpu
from jax.experimental.pallas import tpu_sc as plsc

NSN = 10000
NTN = 10000
TOT = NSN + NTN
HID = 128
EDG = 320000
NB = 64
NLAYER = 3
DEA = 16
EAW = 32
CNT = 16

NC = 2
NSUB = 16
K = 80
NCHUNK = EDG // (NSUB * K)
RPT = 640
NPAD = NSUB * RPT

_sc_mesh = plsc.VectorSubcoreMesh(
    core_axis_name="c", subcore_axis_name="s", num_cores=NC, num_subcores=NSUB)


def _spmm_body(hcat_hbm, gidx_hbm, sidx_hbm, zg_hbm, out_hbm,
               acc, gi, si, rbuf, sem):
    c = lax.axis_index("c")
    s = lax.axis_index("s")
    wid = c * NSUB + s
    pltpu.sync_copy(zg_hbm, acc.at[pl.ds(s * RPT, RPT)])
    plsc.subcore_barrier()
    base = wid * NCHUNK

    def step(j, carry):
        pltpu.sync_copy(gidx_hbm.at[base + j], gi)
        pltpu.sync_copy(sidx_hbm.at[base + j], si)
        pltpu.async_copy(hcat_hbm.at[gi], rbuf, sem).wait()
        pltpu.sync_copy(rbuf, acc.at[si], add=True)
        return carry

    lax.fori_loop(0, NCHUNK, step, 0)
    plsc.subcore_barrier()
    pltpu.sync_copy(acc.at[pl.ds(s * RPT, RPT)],
                    out_hbm.at[pl.ds(c * NPAD + s * RPT, RPT)])


_spmm_call = pl.kernel(
    _spmm_body,
    out_type=jax.ShapeDtypeStruct((2 * NPAD, HID), jnp.float32),
    mesh=_sc_mesh,
    scratch_types=[
        pltpu.VMEM_SHARED((NPAD, HID), jnp.float32),
        pltpu.VMEM((K,), jnp.int32),
        pltpu.VMEM((K,), jnp.int32),
        pltpu.VMEM((K, HID), jnp.float32),
        pltpu.SemaphoreType.DMA,
    ],
)


def _ea_body(ea_hbm, sidx_hbm, zea_hbm, out_hbm, acc, si, ebuf):
    c = lax.axis_index("c")
    s = lax.axis_index("s")
    wid = c * NSUB + s
    pltpu.sync_copy(zea_hbm, acc.at[pl.ds(s * RPT, RPT)])
    plsc.subcore_barrier()
    ebase = s * (NCHUNK * K)
    ibase = wid * NCHUNK

    def step(j, carry):
        pltpu.sync_copy(sidx_hbm.at[ibase + j], si)
        pltpu.sync_copy(ea_hbm.at[pl.ds(ebase + j * K, K)], ebuf)
        pltpu.sync_copy(ebuf, acc.at[si], add=True)
        return carry

    lax.fori_loop(0, NCHUNK, step, 0)
    plsc.subcore_barrier()
    pltpu.sync_copy(acc.at[pl.ds(s * RPT, RPT)],
                    out_hbm.at[pl.ds(c * NPAD + s * RPT, RPT)])


_ea_call = pl.kernel(
    _ea_body,
    out_type=jax.ShapeDtypeStruct((2 * NPAD, EAW), jnp.float32),
    mesh=_sc_mesh,
    compiler_params=pltpu.CompilerParams(use_tc_tiling_on_sc=False),
    scratch_types=[
        pltpu.VMEM_SHARED((NPAD, EAW), jnp.float32),
        pltpu.VMEM((K,), jnp.int32),
        pltpu.VMEM((K, EAW), jnp.float32),
    ],
)


RBLK = 1000
GRID = NSN // RBLK


def _layer_body(hs, ht, gt, gs, eat, eas, bsf, btf,
                ws2t, we2t, wtself, bt, wt2s, we2s, wsself, bs,
                ns_ref, nt_ref, ps_ref, pt_ref):
    f32 = jnp.float32
    i = pl.program_id(0)
    iot = lax.broadcasted_iota(jnp.int32, (1, NB), 1)

    @pl.when(i == 0)
    def _():
        ps_ref[...] = jnp.zeros(ps_ref.shape, f32)
        pt_ref[...] = jnp.zeros(pt_ref.shape, f32)

    ea_t = eat[...]
    inv_t = 1.0 / jnp.maximum(ea_t[:, CNT:CNT + 1], 1.0)
    agg_t = (jnp.dot(gt[...], ws2t[...], preferred_element_type=f32)
             + jnp.dot(ea_t[:, :DEA], we2t[...], preferred_element_type=f32)) * inv_t
    nt = jnp.maximum(
        jnp.dot(ht[...], wtself[...], preferred_element_type=f32)
        + agg_t + bt[...], 0.0)
    nt_ref[...] = nt
    mt = jnp.where(btf[...] == iot, 1.0, 0.0)
    pt_ref[...] += lax.dot_general(
        mt, nt, (((0,), (0,)), ((), ())), preferred_element_type=f32)

    ea_s = eas[...]
    inv_s = 1.0 / jnp.maximum(ea_s[:, CNT:CNT + 1], 1.0)
    agg_s = (jnp.dot(gs[...], wt2s[...], preferred_element_type=f32)
             + jnp.dot(ea_s[:, :DEA], we2s[...], preferred_element_type=f32)) * inv_s
    ns = jnp.maximum(
        jnp.dot(hs[...], wsself[...], preferred_element_type=f32)
        + agg_s + bs[...], 0.0)
    ns_ref[...] = ns
    ms = jnp.where(bsf[...] == iot, 1.0, 0.0)
    ps_ref[...] += lax.dot_general(
        ms, ns, (((0,), (0,)), ((), ())), preferred_element_type=f32)


def _row_spec(w):
    return pl.BlockSpec((RBLK, w), lambda i: (i, 0))


def _full_spec(r, c):
    return pl.BlockSpec((r, c), lambda i: (0, 0))


_layer_call = pl.pallas_call(
    _layer_body,
    grid=(GRID,),
    in_specs=[
        _row_spec(HID), _row_spec(HID),
        _row_spec(HID), _row_spec(HID),
        _row_spec(EAW), _row_spec(EAW),
        _row_spec(1), _row_spec(1),
        _full_spec(HID, HID), _full_spec(DEA, HID), _full_spec(HID, HID),
        _full_spec(1, HID),
        _full_spec(HID, HID), _full_spec(DEA, HID), _full_spec(HID, HID),
        _full_spec(1, HID),
    ],
    out_specs=[
        _row_spec(HID), _row_spec(HID),
        _full_spec(NB, HID), _full_spec(NB, HID),
    ],
    out_shape=[
        jax.ShapeDtypeStruct((NSN, HID), jnp.float32),
        jax.ShapeDtypeStruct((NTN, HID), jnp.float32),
        jax.ShapeDtypeStruct((NB, HID), jnp.float32),
        jax.ShapeDtypeStruct((NB, HID), jnp.float32),
    ],
)


def _head_body(ps0, ps1, ps2, pt0, pt1, pt2, y, wcfg, bcfg, wp, bp, out_ref):
    f32 = jnp.float32
    yemb = jnp.dot(y[...], wcfg[...], preferred_element_type=f32) + bcfg[...]
    acc = jnp.dot(ps0[...], wp[0:128, :], preferred_element_type=f32)
    acc += jnp.dot(ps1[...], wp[128:256, :], preferred_element_type=f32)
    acc += jnp.dot(ps2[...], wp[256:384, :], preferred_element_type=f32)
    acc += jnp.dot(pt0[...], wp[384:512, :], preferred_element_type=f32)
    acc += jnp.dot(pt1[...], wp[512:640, :], preferred_element_type=f32)
    acc += jnp.dot(pt2[...], wp[640:768, :], preferred_element_type=f32)
    acc += jnp.dot(yemb, wp[768:784, :], preferred_element_type=f32)
    out_ref[...] = acc + bp[...]


_head_call = pl.pallas_call(
    _head_body,
    out_shape=jax.ShapeDtypeStruct((NB, 1), jnp.float32),
)


def kernel(x_s, x_t, edge_attr, edge_index, x_s_batch, x_t_batch, y, params):
    f32 = jnp.float32
    src = edge_index[0].astype(jnp.int32)
    dst = edge_index[1].astype(jnp.int32)
    src3 = src.reshape(NSUB, NCHUNK, K)
    dst3 = dst.reshape(NSUB, NCHUNK, K)
    gidx = jnp.concatenate([src3, dst3 + NSN], axis=0).reshape(-1, K)
    sidx = jnp.concatenate([dst3, src3], axis=0).reshape(-1, K)
    ea_pad = jnp.concatenate(
        [edge_attr, jnp.ones((EDG, 1), f32), jnp.zeros((EDG, EAW - DEA - 1), f32)],
        axis=1)
    zg = jnp.zeros((RPT, HID), f32)
    zea = jnp.zeros((RPT, EAW), f32)
    ea2 = _ea_call(ea_pad, sidx, zea)
    ea_t = ea2[:NSN]
    ea_s = ea2[NPAD:NPAD + NSN]
    bs_f = x_s_batch.astype(jnp.int32).reshape(NSN, 1)
    bt_f = x_t_batch.astype(jnp.int32).reshape(NTN, 1)

    h_s, h_t = x_s, x_t
    ps_list, pt_list = [], []
    for l in range(NLAYER):
        p = params['layer%d' % l]
        hcat = jnp.concatenate([h_s, h_t], axis=0)
        g = _spmm_call(hcat, gidx, sidx, zg)
        h_s, h_t, ps, pt = _layer_call(
            h_s, h_t, g[:NSN], g[NPAD:NPAD + NSN], ea_t, ea_s, bs_f, bt_f,
            p['Ws2t'], p['We2t'], p['Wt_self'], p['bt'].reshape(1, HID),
            p['Wt2s'], p['We2s'], p['Ws_self'], p['bs'].reshape(1, HID))
        ps_list.append(ps)
        pt_list.append(pt)

    return _head_call(
        ps_list[0], ps_list[1], ps_list[2],
        pt_list[0], pt_list[1], pt_list[2],
        y, params['W_cfg'], params['b_cfg'].reshape(1, 16),
        params['W_pred'], params['b_pred'].reshape(1, 1))


# speedup vs baseline: 5.0499x; 1.6647x over previous
import functools

import jax
import jax.numpy as jnp
from jax import lax
from jax.experimental import pallas as pl
from jax.experimental.pallas import tpu as pltpu
from jax.experimental.pallas import tpu_sc as plsc

NSN = 10000
NTN = 10000
TOT = NSN + NTN
HID = 128
EDG = 320000
NB = 64
NLAYER = 3
DEA = 16
EAW = 32
CNT = 16

NC = 2
NSUB = 16
K = 80
NCHUNK = EDG // (NSUB * K)
RPT = 640
NPAD = NSUB * RPT

_sc_mesh = plsc.VectorSubcoreMesh(
    core_axis_name="c", subcore_axis_name="s", num_cores=NC, num_subcores=NSUB)


def _spmm_body(hcat_hbm, idx_hbm, zg_hbm, out_hbm,
               acc, ib0, ib1, rb0, rb1, gsem0, gsem1, ssem0, ssem1):
    c = lax.axis_index("c")
    s = lax.axis_index("s")
    wid = c * NSUB + s
    pltpu.sync_copy(zg_hbm, acc.at[pl.ds(s * RPT, RPT)])
    plsc.subcore_barrier()
    base = wid * NCHUNK

    pltpu.sync_copy(idx_hbm.at[base + 0], ib0)
    pltpu.async_copy(hcat_hbm.at[ib0.at[0]], rb0, gsem0)
    pltpu.sync_copy(idx_hbm.at[base + 1], ib1)
    pltpu.async_copy(hcat_hbm.at[ib1.at[0]], rb1, gsem1)

    def step(jj, carry):
        j0 = jj * 2
        pltpu.make_async_copy(hcat_hbm.at[ib0.at[0]], rb0, gsem0).wait()
        pltpu.async_copy(rb0, acc.at[ib0.at[1]], ssem0, add=True)
        pltpu.make_async_copy(hcat_hbm.at[ib1.at[0]], rb1, gsem1).wait()
        pltpu.async_copy(rb1, acc.at[ib1.at[1]], ssem1, add=True)
        pltpu.make_async_copy(rb0, acc.at[ib0.at[1]], ssem0).wait()
        pltpu.sync_copy(idx_hbm.at[base + j0 + 2], ib0)
        pltpu.async_copy(hcat_hbm.at[ib0.at[0]], rb0, gsem0)
        pltpu.make_async_copy(rb1, acc.at[ib1.at[1]], ssem1).wait()
        pltpu.sync_copy(idx_hbm.at[base + j0 + 3], ib1)
        pltpu.async_copy(hcat_hbm.at[ib1.at[0]], rb1, gsem1)
        return carry

    lax.fori_loop(0, (NCHUNK - 2) // 2, step, 0)
    pltpu.make_async_copy(hcat_hbm.at[ib0.at[0]], rb0, gsem0).wait()
    pltpu.async_copy(rb0, acc.at[ib0.at[1]], ssem0, add=True)
    pltpu.make_async_copy(hcat_hbm.at[ib1.at[0]], rb1, gsem1).wait()
    pltpu.async_copy(rb1, acc.at[ib1.at[1]], ssem1, add=True)
    pltpu.make_async_copy(rb0, acc.at[ib0.at[1]], ssem0).wait()
    pltpu.make_async_copy(rb1, acc.at[ib1.at[1]], ssem1).wait()

    plsc.subcore_barrier()
    pltpu.sync_copy(acc.at[pl.ds(s * RPT, RPT)],
                    out_hbm.at[pl.ds(c * NPAD + s * RPT, RPT)])


_spmm_call = pl.kernel(
    _spmm_body,
    out_type=jax.ShapeDtypeStruct((2 * NPAD, HID), jnp.float32),
    mesh=_sc_mesh,
    scratch_types=[
        pltpu.VMEM_SHARED((NPAD, HID), jnp.float32),
        pltpu.VMEM((2, K), jnp.int32),
        pltpu.VMEM((2, K), jnp.int32),
        pltpu.VMEM((K, HID), jnp.float32),
        pltpu.VMEM((K, HID), jnp.float32),
        pltpu.SemaphoreType.DMA,
        pltpu.SemaphoreType.DMA,
        pltpu.SemaphoreType.DMA,
        pltpu.SemaphoreType.DMA,
    ],
)


def _ea_body(ea_hbm, sidx_hbm, zea_hbm, out_hbm, acc, si, ebuf):
    c = lax.axis_index("c")
    s = lax.axis_index("s")
    wid = c * NSUB + s
    pltpu.sync_copy(zea_hbm, acc.at[pl.ds(s * RPT, RPT)])
    plsc.subcore_barrier()
    ebase = s * (NCHUNK * K)
    ibase = wid * NCHUNK

    def step(j, carry):
        pltpu.sync_copy(sidx_hbm.at[ibase + j], si)
        pltpu.sync_copy(ea_hbm.at[pl.ds(ebase + j * K, K)], ebuf)
        pltpu.sync_copy(ebuf, acc.at[si], add=True)
        return carry

    lax.fori_loop(0, NCHUNK, step, 0)
    plsc.subcore_barrier()
    pltpu.sync_copy(acc.at[pl.ds(s * RPT, RPT)],
                    out_hbm.at[pl.ds(c * NPAD + s * RPT, RPT)])


_ea_call = pl.kernel(
    _ea_body,
    out_type=jax.ShapeDtypeStruct((2 * NPAD, EAW), jnp.float32),
    mesh=_sc_mesh,
    compiler_params=pltpu.CompilerParams(use_tc_tiling_on_sc=False),
    scratch_types=[
        pltpu.VMEM_SHARED((NPAD, EAW), jnp.float32),
        pltpu.VMEM((K,), jnp.int32),
        pltpu.VMEM((K, EAW), jnp.float32),
    ],
)


RBLK = 1000
GRID = NSN // RBLK


def _layer_body(hs, ht, gt, gs, eat, eas, bsf, btf,
                ws2t, we2t, wtself, bt, wt2s, we2s, wsself, bs,
                ns_ref, nt_ref, ps_ref, pt_ref):
    f32 = jnp.float32
    i = pl.program_id(0)
    iot = lax.broadcasted_iota(jnp.int32, (1, NB), 1)

    @pl.when(i == 0)
    def _():
        ps_ref[...] = jnp.zeros(ps_ref.shape, f32)
        pt_ref[...] = jnp.zeros(pt_ref.shape, f32)

    ea_t = eat[...]
    inv_t = 1.0 / jnp.maximum(ea_t[:, CNT:CNT + 1], 1.0)
    agg_t = (jnp.dot(gt[...], ws2t[...], preferred_element_type=f32)
             + jnp.dot(ea_t[:, :DEA], we2t[...], preferred_element_type=f32)) * inv_t
    nt = jnp.maximum(
        jnp.dot(ht[...], wtself[...], preferred_element_type=f32)
        + agg_t + bt[...], 0.0)
    nt_ref[...] = nt
    mt = jnp.where(btf[...] == iot, 1.0, 0.0)
    pt_ref[...] += lax.dot_general(
        mt, nt, (((0,), (0,)), ((), ())), preferred_element_type=f32)

    ea_s = eas[...]
    inv_s = 1.0 / jnp.maximum(ea_s[:, CNT:CNT + 1], 1.0)
    agg_s = (jnp.dot(gs[...], wt2s[...], preferred_element_type=f32)
             + jnp.dot(ea_s[:, :DEA], we2s[...], preferred_element_type=f32)) * inv_s
    ns = jnp.maximum(
        jnp.dot(hs[...], wsself[...], preferred_element_type=f32)
        + agg_s + bs[...], 0.0)
    ns_ref[...] = ns
    ms = jnp.where(bsf[...] == iot, 1.0, 0.0)
    ps_ref[...] += lax.dot_general(
        ms, ns, (((0,), (0,)), ((), ())), preferred_element_type=f32)


def _row_spec(w):
    return pl.BlockSpec((RBLK, w), lambda i: (i, 0))


def _full_spec(r, c):
    return pl.BlockSpec((r, c), lambda i: (0, 0))


_layer_call = pl.pallas_call(
    _layer_body,
    grid=(GRID,),
    in_specs=[
        _row_spec(HID), _row_spec(HID),
        _row_spec(HID), _row_spec(HID),
        _row_spec(EAW), _row_spec(EAW),
        _row_spec(1), _row_spec(1),
        _full_spec(HID, HID), _full_spec(DEA, HID), _full_spec(HID, HID),
        _full_spec(1, HID),
        _full_spec(HID, HID), _full_spec(DEA, HID), _full_spec(HID, HID),
        _full_spec(1, HID),
    ],
    out_specs=[
        _row_spec(HID), _row_spec(HID),
        _full_spec(NB, HID), _full_spec(NB, HID),
    ],
    out_shape=[
        jax.ShapeDtypeStruct((NSN, HID), jnp.float32),
        jax.ShapeDtypeStruct((NTN, HID), jnp.float32),
        jax.ShapeDtypeStruct((NB, HID), jnp.float32),
        jax.ShapeDtypeStruct((NB, HID), jnp.float32),
    ],
)


def _head_body(ps0, ps1, ps2, pt0, pt1, pt2, y, wcfg, bcfg, wp, bp, out_ref):
    f32 = jnp.float32
    yemb = jnp.dot(y[...], wcfg[...], preferred_element_type=f32) + bcfg[...]
    acc = jnp.dot(ps0[...], wp[0:128, :], preferred_element_type=f32)
    acc += jnp.dot(ps1[...], wp[128:256, :], preferred_element_type=f32)
    acc += jnp.dot(ps2[...], wp[256:384, :], preferred_element_type=f32)
    acc += jnp.dot(pt0[...], wp[384:512, :], preferred_element_type=f32)
    acc += jnp.dot(pt1[...], wp[512:640, :], preferred_element_type=f32)
    acc += jnp.dot(pt2[...], wp[640:768, :], preferred_element_type=f32)
    acc += jnp.dot(yemb, wp[768:784, :], preferred_element_type=f32)
    out_ref[...] = acc + bp[...]


_head_call = pl.pallas_call(
    _head_body,
    out_shape=jax.ShapeDtypeStruct((NB, 1), jnp.float32),
)


def kernel(x_s, x_t, edge_attr, edge_index, x_s_batch, x_t_batch, y, params):
    f32 = jnp.float32
    src = edge_index[0].astype(jnp.int32)
    dst = edge_index[1].astype(jnp.int32)
    src3 = src.reshape(NSUB, NCHUNK, K)
    dst3 = dst.reshape(NSUB, NCHUNK, K)
    gidx = jnp.concatenate([src3, dst3 + NSN], axis=0).reshape(-1, K)
    sidx = jnp.concatenate([dst3, src3], axis=0).reshape(-1, K)
    idx2 = jnp.stack([gidx, sidx], axis=1)
    ea_pad = jnp.concatenate(
        [edge_attr, jnp.ones((EDG, 1), f32), jnp.zeros((EDG, EAW - DEA - 1), f32)],
        axis=1)
    zg = jnp.zeros((RPT, HID), f32)
    zea = jnp.zeros((RPT, EAW), f32)
    ea2 = _ea_call(ea_pad, sidx, zea)
    ea_t = ea2[:NSN]
    ea_s = ea2[NPAD:NPAD + NSN]
    bs_f = x_s_batch.astype(jnp.int32).reshape(NSN, 1)
    bt_f = x_t_batch.astype(jnp.int32).reshape(NTN, 1)

    h_s, h_t = x_s, x_t
    ps_list, pt_list = [], []
    for l in range(NLAYER):
        p = params['layer%d' % l]
        hcat = jnp.concatenate([h_s, h_t], axis=0)
        g = _spmm_call(hcat, idx2, zg)
        h_s, h_t, ps, pt = _layer_call(
            h_s, h_t, g[:NSN], g[NPAD:NPAD + NSN], ea_t, ea_s, bs_f, bt_f,
            p['Ws2t'], p['We2t'], p['Wt_self'], p['bt'].reshape(1, HID),
            p['Wt2s'], p['We2s'], p['Ws_self'], p['bs'].reshape(1, HID))
        ps_list.append(ps)
        pt_list.append(pt)

    return _head_call(
        ps_list[0], ps_list[1], ps_list[2],
        pt_list[0], pt_list[1], pt_list[2],
        y, params['W_cfg'], params['b_cfg'].reshape(1, 16),
        params['W_pred'], params['b_pred'].reshape(1, 1))


# speedup vs baseline: 5.6227x; 1.1134x over previous
import jax
import jax.numpy as jnp
from jax import lax
from jax.experimental import pallas as pl
from jax.experimental.pallas import tpu as pltpu
from jax.experimental.pallas import tpu_sc as plsc

NSN = 10000
NTN = 10000
TOT = NSN + NTN
HID = 128
EDG = 320000
NB = 64
NLAYER = 3
DEA = 16
EAW = 32
CNT = 16

NC = 2
NSUB = 16
K = 80
NCHUNK = EDG // (NSUB * K)
RPT = 640
NPAD = NSUB * RPT

_sc_mesh = plsc.VectorSubcoreMesh(
    core_axis_name="c", subcore_axis_name="s", num_cores=NC, num_subcores=NSUB)


def _spmm_loop(hcat_hbm, idx_hbm, acc, ib0, ib1, rb0, rb1,
               gsem0, gsem1, ssem0, ssem1, base, extra0=None, extra1=None):
    pltpu.sync_copy(idx_hbm.at[base + 0], ib0)
    pltpu.async_copy(hcat_hbm.at[ib0.at[0]], rb0, gsem0)
    pltpu.sync_copy(idx_hbm.at[base + 1], ib1)
    pltpu.async_copy(hcat_hbm.at[ib1.at[0]], rb1, gsem1)

    def step(jj, carry):
        j0 = jj * 2
        pltpu.make_async_copy(hcat_hbm.at[ib0.at[0]], rb0, gsem0).wait()
        pltpu.async_copy(rb0, acc.at[ib0.at[1]], ssem0, add=True)
        if extra0 is not None:
            extra0[0](j0, ib0)
        pltpu.make_async_copy(hcat_hbm.at[ib1.at[0]], rb1, gsem1).wait()
        pltpu.async_copy(rb1, acc.at[ib1.at[1]], ssem1, add=True)
        if extra1 is not None:
            extra1[0](j0 + 1, ib1)
        pltpu.make_async_copy(rb0, acc.at[ib0.at[1]], ssem0).wait()
        if extra0 is not None:
            extra0[1](ib0)
        pltpu.sync_copy(idx_hbm.at[base + j0 + 2], ib0)
        pltpu.async_copy(hcat_hbm.at[ib0.at[0]], rb0, gsem0)
        pltpu.make_async_copy(rb1, acc.at[ib1.at[1]], ssem1).wait()
        if extra1 is not None:
            extra1[1](ib1)
        pltpu.sync_copy(idx_hbm.at[base + j0 + 3], ib1)
        pltpu.async_copy(hcat_hbm.at[ib1.at[0]], rb1, gsem1)
        return carry

    lax.fori_loop(0, (NCHUNK - 2) // 2, step, 0)
    j0 = NCHUNK - 2
    pltpu.make_async_copy(hcat_hbm.at[ib0.at[0]], rb0, gsem0).wait()
    pltpu.async_copy(rb0, acc.at[ib0.at[1]], ssem0, add=True)
    if extra0 is not None:
        extra0[0](j0, ib0)
    pltpu.make_async_copy(hcat_hbm.at[ib1.at[0]], rb1, gsem1).wait()
    pltpu.async_copy(rb1, acc.at[ib1.at[1]], ssem1, add=True)
    if extra1 is not None:
        extra1[0](j0 + 1, ib1)
    pltpu.make_async_copy(rb0, acc.at[ib0.at[1]], ssem0).wait()
    pltpu.make_async_copy(rb1, acc.at[ib1.at[1]], ssem1).wait()
    if extra0 is not None:
        extra0[1](ib0)
    if extra1 is not None:
        extra1[1](ib1)


def _spmm_body(hcat_hbm, idx_hbm, zg_hbm, out_hbm,
               acc, ib0, ib1, rb0, rb1, gsem0, gsem1, ssem0, ssem1):
    c = lax.axis_index("c")
    s = lax.axis_index("s")
    pltpu.sync_copy(zg_hbm, acc.at[pl.ds(s * RPT, RPT)])
    plsc.subcore_barrier()
    _spmm_loop(hcat_hbm, idx_hbm, acc, ib0, ib1, rb0, rb1,
               gsem0, gsem1, ssem0, ssem1, (c * NSUB + s) * NCHUNK)
    plsc.subcore_barrier()
    pltpu.sync_copy(acc.at[pl.ds(s * RPT, RPT)],
                    out_hbm.at[pl.ds(c * NPAD + s * RPT, RPT)])


_spmm_call = pl.kernel(
    _spmm_body,
    out_type=jax.ShapeDtypeStruct((2 * NPAD, HID), jnp.float32),
    mesh=_sc_mesh,
    scratch_types=[
        pltpu.VMEM_SHARED((NPAD, HID), jnp.float32),
        pltpu.VMEM((2, K), jnp.int32),
        pltpu.VMEM((2, K), jnp.int32),
        pltpu.VMEM((K, HID), jnp.float32),
        pltpu.VMEM((K, HID), jnp.float32),
        pltpu.SemaphoreType.DMA,
        pltpu.SemaphoreType.DMA,
        pltpu.SemaphoreType.DMA,
        pltpu.SemaphoreType.DMA,
    ],
)


def _spmm_ea_body(hcat_hbm, idx_hbm, ea_hbm, zg_hbm, zea_hbm, out_hbm, oea_hbm,
                  acc, eacc, ib0, ib1, rb0, rb1, eb0, eb1,
                  gsem0, gsem1, ssem0, ssem1, esem0, esem1):
    c = lax.axis_index("c")
    s = lax.axis_index("s")
    pltpu.sync_copy(zg_hbm, acc.at[pl.ds(s * RPT, RPT)])
    pltpu.sync_copy(zea_hbm, eacc.at[pl.ds(s * RPT, RPT)])
    plsc.subcore_barrier()
    ebase = s * (NCHUNK * K)

    def issue(j, ib, eb, esem):
        pltpu.sync_copy(ea_hbm.at[pl.ds(ebase + j * K, K)], eb)
        pltpu.async_copy(eb, eacc.at[ib.at[1]], esem, add=True)

    def drain(ib, eb, esem):
        pltpu.make_async_copy(eb, eacc.at[ib.at[1]], esem).wait()

    _spmm_loop(hcat_hbm, idx_hbm, acc, ib0, ib1, rb0, rb1,
               gsem0, gsem1, ssem0, ssem1, (c * NSUB + s) * NCHUNK,
               extra0=(lambda j, ib: issue(j, ib, eb0, esem0),
                       lambda ib: drain(ib, eb0, esem0)),
               extra1=(lambda j, ib: issue(j, ib, eb1, esem1),
                       lambda ib: drain(ib, eb1, esem1)))
    plsc.subcore_barrier()
    pltpu.sync_copy(acc.at[pl.ds(s * RPT, RPT)],
                    out_hbm.at[pl.ds(c * NPAD + s * RPT, RPT)])
    pltpu.sync_copy(eacc.at[pl.ds(s * RPT, RPT)],
                    oea_hbm.at[pl.ds(c * NPAD + s * RPT, RPT)])


_spmm_ea_call = pl.kernel(
    _spmm_ea_body,
    out_type=(jax.ShapeDtypeStruct((2 * NPAD, HID), jnp.float32),
              jax.ShapeDtypeStruct((2 * NPAD, EAW), jnp.float32)),
    mesh=_sc_mesh,
    compiler_params=pltpu.CompilerParams(use_tc_tiling_on_sc=False),
    scratch_types=[
        pltpu.VMEM_SHARED((NPAD, HID), jnp.float32),
        pltpu.VMEM_SHARED((NPAD, EAW), jnp.float32),
        pltpu.VMEM((2, K), jnp.int32),
        pltpu.VMEM((2, K), jnp.int32),
        pltpu.VMEM((K, HID), jnp.float32),
        pltpu.VMEM((K, HID), jnp.float32),
        pltpu.VMEM((K, EAW), jnp.float32),
        pltpu.VMEM((K, EAW), jnp.float32),
        pltpu.SemaphoreType.DMA,
        pltpu.SemaphoreType.DMA,
        pltpu.SemaphoreType.DMA,
        pltpu.SemaphoreType.DMA,
        pltpu.SemaphoreType.DMA,
        pltpu.SemaphoreType.DMA,
    ],
)


RBLK = 1000
GRID = NSN // RBLK


def _layer_body(h2, g2, ea2, bids, wx, we, wself, bias,
                hout_ref, pool_ref):
    f32 = jnp.float32
    i = pl.program_id(1)
    iot = lax.broadcasted_iota(jnp.int32, (1, NB), 1)

    @pl.when(i == 0)
    def _():
        pool_ref[...] = jnp.zeros(pool_ref.shape, f32)

    ea = ea2[0]
    inv = 1.0 / jnp.maximum(ea[:, CNT:CNT + 1], 1.0)
    agg = (jnp.dot(g2[0], wx[0], preferred_element_type=f32)
           + jnp.dot(ea[:, :DEA], we[0], preferred_element_type=f32)) * inv
    nh = jnp.maximum(
        jnp.dot(h2[0], wself[0], preferred_element_type=f32)
        + agg + bias[0], 0.0)
    hout_ref[0] = nh
    mask = jnp.where(bids[0] == iot, 1.0, 0.0)
    pool_ref[0] += lax.dot_general(
        mask, nh, (((0,), (0,)), ((), ())), preferred_element_type=f32)


_layer_call = pl.pallas_call(
    _layer_body,
    grid=(2, GRID),
    in_specs=[
        pl.BlockSpec((1, RBLK, HID), lambda b, i: (b, i, 0)),
        pl.BlockSpec((1, RBLK, HID), lambda b, i: (b, i, 0)),
        pl.BlockSpec((1, RBLK, EAW), lambda b, i: (b, i, 0)),
        pl.BlockSpec((1, RBLK, 1), lambda b, i: (b, i, 0)),
        pl.BlockSpec((1, HID, HID), lambda b, i: (b, 0, 0)),
        pl.BlockSpec((1, DEA, HID), lambda b, i: (b, 0, 0)),
        pl.BlockSpec((1, HID, HID), lambda b, i: (b, 0, 0)),
        pl.BlockSpec((1, 1, HID), lambda b, i: (b, 0, 0)),
    ],
    out_specs=[
        pl.BlockSpec((1, RBLK, HID), lambda b, i: (b, i, 0)),
        pl.BlockSpec((1, NB, HID), lambda b, i: (b, 0, 0)),
    ],
    out_shape=[
        jax.ShapeDtypeStruct((2, NSN, HID), jnp.float32),
        jax.ShapeDtypeStruct((2, NB, HID), jnp.float32),
    ],
)


def _head_body(p0, p1, p2, y, wcfg, bcfg, wp, bp, out_ref):
    f32 = jnp.float32
    yemb = jnp.dot(y[...], wcfg[...], preferred_element_type=f32) + bcfg[...]
    acc = jnp.dot(p0[1], wp[0:128, :], preferred_element_type=f32)
    acc += jnp.dot(p1[1], wp[128:256, :], preferred_element_type=f32)
    acc += jnp.dot(p2[1], wp[256:384, :], preferred_element_type=f32)
    acc += jnp.dot(p0[0], wp[384:512, :], preferred_element_type=f32)
    acc += jnp.dot(p1[0], wp[512:640, :], preferred_element_type=f32)
    acc += jnp.dot(p2[0], wp[640:768, :], preferred_element_type=f32)
    acc += jnp.dot(yemb, wp[768:784, :], preferred_element_type=f32)
    out_ref[...] = acc + bp[...]


_head_call = pl.pallas_call(
    _head_body,
    out_shape=jax.ShapeDtypeStruct((NB, 1), jnp.float32),
)


def kernel(x_s, x_t, edge_attr, edge_index, x_s_batch, x_t_batch, y, params):
    f32 = jnp.float32
    src = edge_index[0].astype(jnp.int32)
    dst = edge_index[1].astype(jnp.int32)
    src3 = src.reshape(NSUB, NCHUNK, K)
    dst3 = dst.reshape(NSUB, NCHUNK, K)
    gidx = jnp.concatenate([src3 + NSN, dst3], axis=0).reshape(-1, K)
    sidx = jnp.concatenate([dst3, src3], axis=0).reshape(-1, K)
    idx2 = jnp.stack([gidx, sidx], axis=1)
    ea_pad = jnp.concatenate(
        [edge_attr, jnp.ones((EDG, 1), f32), jnp.zeros((EDG, EAW - DEA - 1), f32)],
        axis=1)
    zg = jnp.zeros((RPT, HID), f32)
    zea = jnp.zeros((RPT, EAW), f32)
    bids = jnp.stack([x_t_batch, x_s_batch]).astype(jnp.int32).reshape(2, NSN, 1)

    h2 = jnp.stack([x_t, x_s])
    pools = []
    ea3 = None
    for l in range(NLAYER):
        p = params['layer%d' % l]
        hcat = h2.reshape(TOT, HID)
        if l == 0:
            g, eao = _spmm_ea_call(hcat, idx2, ea_pad, zg, zea)
            ea3 = eao.reshape(2, NPAD, EAW)
        else:
            g = _spmm_call(hcat, idx2, zg)
        h2, pool = _layer_call(
            h2, g.reshape(2, NPAD, HID), ea3,
            bids,
            jnp.stack([p['Ws2t'], p['Wt2s']]),
            jnp.stack([p['We2t'], p['We2s']]),
            jnp.stack([p['Wt_self'], p['Ws_self']]),
            jnp.stack([p['bt'], p['bs']]).reshape(2, 1, HID))
        pools.append(pool)

    return _head_call(
        pools[0], pools[1], pools[2],
        y, params['W_cfg'], params['b_cfg'].reshape(1, 16),
        params['W_pred'], params['b_pred'].reshape(1, 1))


# speedup vs baseline: 5.7013x; 1.0140x over previous
import jax
import jax.numpy as jnp
from jax import lax
from jax.experimental import pallas as pl
from jax.experimental.pallas import tpu as pltpu
from jax.experimental.pallas import tpu_sc as plsc

NSN = 10000
NTN = 10000
TOT = NSN + NTN
HID = 128
EDG = 320000
NB = 64
NLAYER = 3
DEA = 16
EAW = 32
CNT = 16

NC = 2
NSUB = 16
K = 80
NCHUNK = EDG // (NSUB * K)
RPT = 640
NPAD = NSUB * RPT

_sc_mesh = plsc.VectorSubcoreMesh(
    core_axis_name="c", subcore_axis_name="s", num_cores=NC, num_subcores=NSUB)


def _spmm_loop(hcat_hbm, idx_hbm, acc, ib0, ib1, rb0, rb1,
               gsem0, gsem1, ssem0, ssem1, base, extra0=None, extra1=None):
    pltpu.sync_copy(idx_hbm.at[base + 0], ib0)
    pltpu.async_copy(hcat_hbm.at[ib0.at[0]], rb0, gsem0)
    pltpu.sync_copy(idx_hbm.at[base + 1], ib1)
    pltpu.async_copy(hcat_hbm.at[ib1.at[0]], rb1, gsem1)

    def step(jj, carry):
        j0 = jj * 2
        pltpu.make_async_copy(hcat_hbm.at[ib0.at[0]], rb0, gsem0).wait()
        pltpu.async_copy(rb0, acc.at[ib0.at[1]], ssem0, add=True)
        if extra0 is not None:
            extra0[0](j0, ib0)
        pltpu.make_async_copy(hcat_hbm.at[ib1.at[0]], rb1, gsem1).wait()
        pltpu.async_copy(rb1, acc.at[ib1.at[1]], ssem1, add=True)
        if extra1 is not None:
            extra1[0](j0 + 1, ib1)
        pltpu.make_async_copy(rb0, acc.at[ib0.at[1]], ssem0).wait()
        if extra0 is not None:
            extra0[1](ib0)
        pltpu.sync_copy(idx_hbm.at[base + j0 + 2], ib0)
        pltpu.async_copy(hcat_hbm.at[ib0.at[0]], rb0, gsem0)
        pltpu.make_async_copy(rb1, acc.at[ib1.at[1]], ssem1).wait()
        if extra1 is not None:
            extra1[1](ib1)
        pltpu.sync_copy(idx_hbm.at[base + j0 + 3], ib1)
        pltpu.async_copy(hcat_hbm.at[ib1.at[0]], rb1, gsem1)
        return carry

    lax.fori_loop(0, (NCHUNK - 2) // 2, step, 0)
    j0 = NCHUNK - 2
    pltpu.make_async_copy(hcat_hbm.at[ib0.at[0]], rb0, gsem0).wait()
    pltpu.async_copy(rb0, acc.at[ib0.at[1]], ssem0, add=True)
    if extra0 is not None:
        extra0[0](j0, ib0)
    pltpu.make_async_copy(hcat_hbm.at[ib1.at[0]], rb1, gsem1).wait()
    pltpu.async_copy(rb1, acc.at[ib1.at[1]], ssem1, add=True)
    if extra1 is not None:
        extra1[0](j0 + 1, ib1)
    pltpu.make_async_copy(rb0, acc.at[ib0.at[1]], ssem0).wait()
    pltpu.make_async_copy(rb1, acc.at[ib1.at[1]], ssem1).wait()
    if extra0 is not None:
        extra0[1](ib0)
    if extra1 is not None:
        extra1[1](ib1)


def _spmm_body(hcat_hbm, idx_hbm, zg_hbm, out_hbm,
               acc, ib0, ib1, rb0, rb1, gsem0, gsem1, ssem0, ssem1):
    c = lax.axis_index("c")
    s = lax.axis_index("s")
    pltpu.sync_copy(zg_hbm, acc.at[pl.ds(s * RPT, RPT)])
    plsc.subcore_barrier()
    _spmm_loop(hcat_hbm, idx_hbm, acc, ib0, ib1, rb0, rb1,
               gsem0, gsem1, ssem0, ssem1, (c * NSUB + s) * NCHUNK)
    plsc.subcore_barrier()
    pltpu.sync_copy(acc.at[pl.ds(s * RPT, RPT)],
                    out_hbm.at[pl.ds(c * NPAD + s * RPT, RPT)])


_spmm_call = pl.kernel(
    _spmm_body,
    out_type=jax.ShapeDtypeStruct((2 * NPAD, HID), jnp.float32),
    mesh=_sc_mesh,
    scratch_types=[
        pltpu.VMEM_SHARED((NPAD, HID), jnp.float32),
        pltpu.VMEM((2, K), jnp.int32),
        pltpu.VMEM((2, K), jnp.int32),
        pltpu.VMEM((K, HID), jnp.float32),
        pltpu.VMEM((K, HID), jnp.float32),
        pltpu.SemaphoreType.DMA,
        pltpu.SemaphoreType.DMA,
        pltpu.SemaphoreType.DMA,
        pltpu.SemaphoreType.DMA,
    ],
)


def _spmm_ea_body(hcat_hbm, idx_hbm, ea_hbm, zg_hbm, zea_hbm, out_hbm, oea_hbm,
                  acc, eacc, ib0, ib1, rb0, rb1, eb0, eb1,
                  gsem0, gsem1, ssem0, ssem1, esem0, esem1):
    c = lax.axis_index("c")
    s = lax.axis_index("s")
    pltpu.sync_copy(zg_hbm, acc.at[pl.ds(s * RPT, RPT)])
    pltpu.sync_copy(zea_hbm, eacc.at[pl.ds(s * RPT, RPT)])
    tail = jnp.where(lax.broadcasted_iota(jnp.int32, (16,), 0) == 0, 1.0, 0.0)

    def fill(r, carry):
        eb0[r, pl.ds(DEA, 16)] = tail
        eb1[r, pl.ds(DEA, 16)] = tail
        return carry

    lax.fori_loop(0, K, fill, 0)
    plsc.subcore_barrier()
    ebase = s * (NCHUNK * K)

    def issue(j, ib, eb, esem):
        pltpu.sync_copy(ea_hbm.at[pl.ds(ebase + j * K, K)], eb.at[:, pl.ds(0, DEA)])
        pltpu.async_copy(eb, eacc.at[ib.at[1]], esem, add=True)

    def drain(ib, eb, esem):
        pltpu.make_async_copy(eb, eacc.at[ib.at[1]], esem).wait()

    _spmm_loop(hcat_hbm, idx_hbm, acc, ib0, ib1, rb0, rb1,
               gsem0, gsem1, ssem0, ssem1, (c * NSUB + s) * NCHUNK,
               extra0=(lambda j, ib: issue(j, ib, eb0, esem0),
                       lambda ib: drain(ib, eb0, esem0)),
               extra1=(lambda j, ib: issue(j, ib, eb1, esem1),
                       lambda ib: drain(ib, eb1, esem1)))
    plsc.subcore_barrier()
    pltpu.sync_copy(acc.at[pl.ds(s * RPT, RPT)],
                    out_hbm.at[pl.ds(c * NPAD + s * RPT, RPT)])
    pltpu.sync_copy(eacc.at[pl.ds(s * RPT, RPT)],
                    oea_hbm.at[pl.ds(c * NPAD + s * RPT, RPT)])


_spmm_ea_call = pl.kernel(
    _spmm_ea_body,
    out_type=(jax.ShapeDtypeStruct((2 * NPAD, HID), jnp.float32),
              jax.ShapeDtypeStruct((2 * NPAD, EAW), jnp.float32)),
    mesh=_sc_mesh,
    compiler_params=pltpu.CompilerParams(use_tc_tiling_on_sc=False),
    scratch_types=[
        pltpu.VMEM_SHARED((NPAD, HID), jnp.float32),
        pltpu.VMEM_SHARED((NPAD, EAW), jnp.float32),
        pltpu.VMEM((2, K), jnp.int32),
        pltpu.VMEM((2, K), jnp.int32),
        pltpu.VMEM((K, HID), jnp.float32),
        pltpu.VMEM((K, HID), jnp.float32),
        pltpu.VMEM((K, EAW), jnp.float32),
        pltpu.VMEM((K, EAW), jnp.float32),
        pltpu.SemaphoreType.DMA,
        pltpu.SemaphoreType.DMA,
        pltpu.SemaphoreType.DMA,
        pltpu.SemaphoreType.DMA,
        pltpu.SemaphoreType.DMA,
        pltpu.SemaphoreType.DMA,
    ],
)


RBLK = 2000
GRID = NSN // RBLK


def _layer_body(h2, g2, ea2, bids, wx, we, wself, bias,
                hout_ref, pool_ref):
    f32 = jnp.float32
    i = pl.program_id(1)
    iot = lax.broadcasted_iota(jnp.int32, (1, NB), 1)

    @pl.when(i == 0)
    def _():
        pool_ref[...] = jnp.zeros(pool_ref.shape, f32)

    ea = ea2[0]
    inv = 1.0 / jnp.maximum(ea[:, CNT:CNT + 1], 1.0)
    agg = (jnp.dot(g2[0], wx[0], preferred_element_type=f32)
           + jnp.dot(ea[:, :DEA], we[0], preferred_element_type=f32)) * inv
    nh = jnp.maximum(
        jnp.dot(h2[0], wself[0], preferred_element_type=f32)
        + agg + bias[0], 0.0)
    hout_ref[0] = nh
    mask = jnp.where(bids[0] == iot, 1.0, 0.0)
    pool_ref[0] += lax.dot_general(
        mask, nh, (((0,), (0,)), ((), ())), preferred_element_type=f32)


_layer_call = pl.pallas_call(
    _layer_body,
    grid=(2, GRID),
    in_specs=[
        pl.BlockSpec((1, RBLK, HID), lambda b, i: (b, i, 0)),
        pl.BlockSpec((1, RBLK, HID), lambda b, i: (b, i, 0)),
        pl.BlockSpec((1, RBLK, EAW), lambda b, i: (b, i, 0)),
        pl.BlockSpec((1, RBLK, 1), lambda b, i: (b, i, 0)),
        pl.BlockSpec((1, HID, HID), lambda b, i: (b, 0, 0)),
        pl.BlockSpec((1, DEA, HID), lambda b, i: (b, 0, 0)),
        pl.BlockSpec((1, HID, HID), lambda b, i: (b, 0, 0)),
        pl.BlockSpec((1, 1, HID), lambda b, i: (b, 0, 0)),
    ],
    out_specs=[
        pl.BlockSpec((1, RBLK, HID), lambda b, i: (b, i, 0)),
        pl.BlockSpec((1, NB, HID), lambda b, i: (b, 0, 0)),
    ],
    out_shape=[
        jax.ShapeDtypeStruct((2, NSN, HID), jnp.float32),
        jax.ShapeDtypeStruct((2, NB, HID), jnp.float32),
    ],
)


def _head_body(p0, p1, p2, y, wcfg, bcfg, wp, bp, out_ref):
    f32 = jnp.float32
    yemb = jnp.dot(y[...], wcfg[...], preferred_element_type=f32) + bcfg[...]
    acc = jnp.dot(p0[1], wp[0:128, :], preferred_element_type=f32)
    acc += jnp.dot(p1[1], wp[128:256, :], preferred_element_type=f32)
    acc += jnp.dot(p2[1], wp[256:384, :], preferred_element_type=f32)
    acc += jnp.dot(p0[0], wp[384:512, :], preferred_element_type=f32)
    acc += jnp.dot(p1[0], wp[512:640, :], preferred_element_type=f32)
    acc += jnp.dot(p2[0], wp[640:768, :], preferred_element_type=f32)
    acc += jnp.dot(yemb, wp[768:784, :], preferred_element_type=f32)
    out_ref[...] = acc + bp[...]


_head_call = pl.pallas_call(
    _head_body,
    out_shape=jax.ShapeDtypeStruct((NB, 1), jnp.float32),
)


def kernel(x_s, x_t, edge_attr, edge_index, x_s_batch, x_t_batch, y, params):
    f32 = jnp.float32
    src = edge_index[0].astype(jnp.int32)
    dst = edge_index[1].astype(jnp.int32)
    src3 = src.reshape(NSUB, NCHUNK, K)
    dst3 = dst.reshape(NSUB, NCHUNK, K)
    gidx = jnp.concatenate([src3 + NSN, dst3], axis=0).reshape(-1, K)
    sidx = jnp.concatenate([dst3, src3], axis=0).reshape(-1, K)
    idx2 = jnp.stack([gidx, sidx], axis=1)
    zg = jnp.zeros((RPT, HID), f32)
    zea = jnp.zeros((RPT, EAW), f32)
    bids = jnp.stack([x_t_batch, x_s_batch]).astype(jnp.int32).reshape(2, NSN, 1)

    h2 = jnp.stack([x_t, x_s])
    pools = []
    ea3 = None
    for l in range(NLAYER):
        p = params['layer%d' % l]
        hcat = h2.reshape(TOT, HID)
        if l == 0:
            g, eao = _spmm_ea_call(hcat, idx2, edge_attr, zg, zea)
            ea3 = eao.reshape(2, NPAD, EAW)
        else:
            g = _spmm_call(hcat, idx2, zg)
        h2, pool = _layer_call(
            h2, g.reshape(2, NPAD, HID), ea3,
            bids,
            jnp.stack([p['Ws2t'], p['Wt2s']]),
            jnp.stack([p['We2t'], p['We2s']]),
            jnp.stack([p['Wt_self'], p['Ws_self']]),
            jnp.stack([p['bt'], p['bs']]).reshape(2, 1, HID))
        pools.append(pool)

    return _head_call(
        pools[0], pools[1], pools[2],
        y, params['W_cfg'], params['b_cfg'].reshape(1, 16),
        params['W_pred'], params['b_pred'].reshape(1, 1))


# speedup vs baseline: 5.9289x; 1.0399x over previous
import jax
import jax.numpy as jnp
from jax import lax
from jax.experimental import pallas as pl
from jax.experimental.pallas import tpu as pltpu
from jax.experimental.pallas import tpu_sc as plsc

NSN = 10000
NTN = 10000
TOT = NSN + NTN
HID = 128
EDG = 320000
NB = 64
NLAYER = 3
DEA = 16
EAW = 32
CNT = 16

NC = 2
NSUB = 16
K = 80
NCHUNK = EDG // (NSUB * K)
RPT = 640
NPAD = NSUB * RPT

_sc_mesh = plsc.VectorSubcoreMesh(
    core_axis_name="c", subcore_axis_name="s", num_cores=NC, num_subcores=NSUB)


def _spmm_loop(hcat_hbm, idx_hbm, acc, ib0, ib1, rb0, rb1,
               gsem0, gsem1, ssem0, ssem1, base, extra0=None, extra1=None):
    pltpu.sync_copy(idx_hbm.at[base + 0], ib0)
    pltpu.async_copy(hcat_hbm.at[ib0.at[0]], rb0, gsem0)
    pltpu.sync_copy(idx_hbm.at[base + 1], ib1)
    pltpu.async_copy(hcat_hbm.at[ib1.at[0]], rb1, gsem1)

    def step(jj, carry):
        j0 = jj * 2
        pltpu.make_async_copy(hcat_hbm.at[ib0.at[0]], rb0, gsem0).wait()
        pltpu.async_copy(rb0, acc.at[ib0.at[1]], ssem0, add=True)
        if extra0 is not None:
            extra0[0](j0, ib0)
        pltpu.make_async_copy(hcat_hbm.at[ib1.at[0]], rb1, gsem1).wait()
        pltpu.async_copy(rb1, acc.at[ib1.at[1]], ssem1, add=True)
        if extra1 is not None:
            extra1[0](j0 + 1, ib1)
        pltpu.make_async_copy(rb0, acc.at[ib0.at[1]], ssem0).wait()
        if extra0 is not None:
            extra0[1](ib0)
        pltpu.sync_copy(idx_hbm.at[base + j0 + 2], ib0)
        pltpu.async_copy(hcat_hbm.at[ib0.at[0]], rb0, gsem0)
        pltpu.make_async_copy(rb1, acc.at[ib1.at[1]], ssem1).wait()
        if extra1 is not None:
            extra1[1](ib1)
        pltpu.sync_copy(idx_hbm.at[base + j0 + 3], ib1)
        pltpu.async_copy(hcat_hbm.at[ib1.at[0]], rb1, gsem1)
        return carry

    lax.fori_loop(0, (NCHUNK - 2) // 2, step, 0)
    j0 = NCHUNK - 2
    pltpu.make_async_copy(hcat_hbm.at[ib0.at[0]], rb0, gsem0).wait()
    pltpu.async_copy(rb0, acc.at[ib0.at[1]], ssem0, add=True)
    if extra0 is not None:
        extra0[0](j0, ib0)
    pltpu.make_async_copy(hcat_hbm.at[ib1.at[0]], rb1, gsem1).wait()
    pltpu.async_copy(rb1, acc.at[ib1.at[1]], ssem1, add=True)
    if extra1 is not None:
        extra1[0](j0 + 1, ib1)
    pltpu.make_async_copy(rb0, acc.at[ib0.at[1]], ssem0).wait()
    pltpu.make_async_copy(rb1, acc.at[ib1.at[1]], ssem1).wait()
    if extra0 is not None:
        extra0[1](ib0)
    if extra1 is not None:
        extra1[1](ib1)


def _spmm_body(hcat_hbm, idx_hbm, zg_hbm, out_hbm,
               acc, ib0, ib1, rb0, rb1, gsem0, gsem1, ssem0, ssem1):
    c = lax.axis_index("c")
    s = lax.axis_index("s")
    pltpu.sync_copy(zg_hbm, acc.at[pl.ds(s * RPT, RPT)])
    plsc.subcore_barrier()
    _spmm_loop(hcat_hbm, idx_hbm, acc, ib0, ib1, rb0, rb1,
               gsem0, gsem1, ssem0, ssem1, (c * NSUB + s) * NCHUNK)
    plsc.subcore_barrier()
    pltpu.sync_copy(acc.at[pl.ds(s * RPT, RPT)],
                    out_hbm.at[pl.ds(c * NPAD + s * RPT, RPT)])


_spmm_call = pl.kernel(
    _spmm_body,
    out_type=jax.ShapeDtypeStruct((2 * NPAD, HID), jnp.float32),
    mesh=_sc_mesh,
    scratch_types=[
        pltpu.VMEM_SHARED((NPAD, HID), jnp.float32),
        pltpu.VMEM((2, K), jnp.int32),
        pltpu.VMEM((2, K), jnp.int32),
        pltpu.VMEM((K, HID), jnp.float32),
        pltpu.VMEM((K, HID), jnp.float32),
        pltpu.SemaphoreType.DMA,
        pltpu.SemaphoreType.DMA,
        pltpu.SemaphoreType.DMA,
        pltpu.SemaphoreType.DMA,
    ],
)


def _spmm_ea_body(hcat_hbm, idx_hbm, ea_hbm, zg_hbm, zea_hbm,
                  out_hbm, oea_hbm, ocnt_hbm,
                  acc, eacc, cacc, ib0, ib1, rb0, rb1, eb0, eb1, ones,
                  gsem0, gsem1, ssem0, ssem1, esem0, esem1, csem0, csem1):
    c = lax.axis_index("c")
    s = lax.axis_index("s")
    pltpu.sync_copy(zg_hbm, acc.at[pl.ds(s * RPT, RPT)])
    pltpu.sync_copy(zea_hbm, eacc.at[pl.ds(s * RPT, RPT)])
    pltpu.sync_copy(zea_hbm, cacc.at[pl.ds(s * RPT, RPT)])
    one_row = jnp.zeros((16,), jnp.float32) + 1.0

    def fill(r, carry):
        ones[r, pl.ds(0, 16)] = one_row
        return carry

    lax.fori_loop(0, K, fill, 0)
    plsc.subcore_barrier()
    ebase = s * (NCHUNK * K)

    def issue(j, ib, eb, esem, csem):
        pltpu.sync_copy(ea_hbm.at[pl.ds(ebase + j * K, K)], eb)
        pltpu.async_copy(eb, eacc.at[ib.at[1]], esem, add=True)
        pltpu.async_copy(ones, cacc.at[ib.at[1]], csem, add=True)

    def drain(ib, eb, esem, csem):
        pltpu.make_async_copy(eb, eacc.at[ib.at[1]], esem).wait()
        pltpu.make_async_copy(ones, cacc.at[ib.at[1]], csem).wait()

    _spmm_loop(hcat_hbm, idx_hbm, acc, ib0, ib1, rb0, rb1,
               gsem0, gsem1, ssem0, ssem1, (c * NSUB + s) * NCHUNK,
               extra0=(lambda j, ib: issue(j, ib, eb0, esem0, csem0),
                       lambda ib: drain(ib, eb0, esem0, csem0)),
               extra1=(lambda j, ib: issue(j, ib, eb1, esem1, csem1),
                       lambda ib: drain(ib, eb1, esem1, csem1)))
    plsc.subcore_barrier()
    pltpu.sync_copy(acc.at[pl.ds(s * RPT, RPT)],
                    out_hbm.at[pl.ds(c * NPAD + s * RPT, RPT)])
    pltpu.sync_copy(eacc.at[pl.ds(s * RPT, RPT)],
                    oea_hbm.at[pl.ds(c * NPAD + s * RPT, RPT)])
    pltpu.sync_copy(cacc.at[pl.ds(s * RPT, RPT)],
                    ocnt_hbm.at[pl.ds(c * NPAD + s * RPT, RPT)])


_spmm_ea_call = pl.kernel(
    _spmm_ea_body,
    out_type=(jax.ShapeDtypeStruct((2 * NPAD, HID), jnp.float32),
              jax.ShapeDtypeStruct((2 * NPAD, DEA), jnp.float32),
              jax.ShapeDtypeStruct((2 * NPAD, DEA), jnp.float32)),
    mesh=_sc_mesh,
    compiler_params=pltpu.CompilerParams(use_tc_tiling_on_sc=False),
    scratch_types=[
        pltpu.VMEM_SHARED((NPAD, HID), jnp.float32),
        pltpu.VMEM_SHARED((NPAD, DEA), jnp.float32),
        pltpu.VMEM_SHARED((NPAD, DEA), jnp.float32),
        pltpu.VMEM((2, K), jnp.int32),
        pltpu.VMEM((2, K), jnp.int32),
        pltpu.VMEM((K, HID), jnp.float32),
        pltpu.VMEM((K, HID), jnp.float32),
        pltpu.VMEM((K, DEA), jnp.float32),
        pltpu.VMEM((K, DEA), jnp.float32),
        pltpu.VMEM((K, DEA), jnp.float32),
        pltpu.SemaphoreType.DMA,
        pltpu.SemaphoreType.DMA,
        pltpu.SemaphoreType.DMA,
        pltpu.SemaphoreType.DMA,
        pltpu.SemaphoreType.DMA,
        pltpu.SemaphoreType.DMA,
        pltpu.SemaphoreType.DMA,
        pltpu.SemaphoreType.DMA,
    ],
)


RBLK = 2000
GRID = NSN // RBLK


def _layer_body(h2, g2, ea2, cnt2, bids, wx, we, wself, bias,
                hout_ref, pool_ref):
    f32 = jnp.float32
    i = pl.program_id(1)
    iot = lax.broadcasted_iota(jnp.int32, (1, NB), 1)

    @pl.when(i == 0)
    def _():
        pool_ref[...] = jnp.zeros(pool_ref.shape, f32)

    inv = 1.0 / jnp.maximum(cnt2[0][:, 0:1], 1.0)
    agg = (jnp.dot(g2[0], wx[0], preferred_element_type=f32)
           + jnp.dot(ea2[0], we[0], preferred_element_type=f32)) * inv
    nh = jnp.maximum(
        jnp.dot(h2[0], wself[0], preferred_element_type=f32)
        + agg + bias[0], 0.0)
    hout_ref[0] = nh
    mask = jnp.where(bids[0] == iot, 1.0, 0.0)
    pool_ref[0] += lax.dot_general(
        mask, nh, (((0,), (0,)), ((), ())), preferred_element_type=f32)


_layer_call = pl.pallas_call(
    _layer_body,
    grid=(2, GRID),
    in_specs=[
        pl.BlockSpec((1, RBLK, HID), lambda b, i: (b, i, 0)),
        pl.BlockSpec((1, RBLK, HID), lambda b, i: (b, i, 0)),
        pl.BlockSpec((1, RBLK, DEA), lambda b, i: (b, i, 0)),
        pl.BlockSpec((1, RBLK, DEA), lambda b, i: (b, i, 0)),
        pl.BlockSpec((1, RBLK, 1), lambda b, i: (b, i, 0)),
        pl.BlockSpec((1, HID, HID), lambda b, i: (b, 0, 0)),
        pl.BlockSpec((1, DEA, HID), lambda b, i: (b, 0, 0)),
        pl.BlockSpec((1, HID, HID), lambda b, i: (b, 0, 0)),
        pl.BlockSpec((1, 1, HID), lambda b, i: (b, 0, 0)),
    ],
    out_specs=[
        pl.BlockSpec((1, RBLK, HID), lambda b, i: (b, i, 0)),
        pl.BlockSpec((1, NB, HID), lambda b, i: (b, 0, 0)),
    ],
    out_shape=[
        jax.ShapeDtypeStruct((2, NSN, HID), jnp.float32),
        jax.ShapeDtypeStruct((2, NB, HID), jnp.float32),
    ],
)


def _head_body(p0, p1, p2, y, wcfg, bcfg, wp, bp, out_ref):
    f32 = jnp.float32
    yemb = jnp.dot(y[...], wcfg[...], preferred_element_type=f32) + bcfg[...]
    acc = jnp.dot(p0[1], wp[0:128, :], preferred_element_type=f32)
    acc += jnp.dot(p1[1], wp[128:256, :], preferred_element_type=f32)
    acc += jnp.dot(p2[1], wp[256:384, :], preferred_element_type=f32)
    acc += jnp.dot(p0[0], wp[384:512, :], preferred_element_type=f32)
    acc += jnp.dot(p1[0], wp[512:640, :], preferred_element_type=f32)
    acc += jnp.dot(p2[0], wp[640:768, :], preferred_element_type=f32)
    acc += jnp.dot(yemb, wp[768:784, :], preferred_element_type=f32)
    out_ref[...] = acc + bp[...]


_head_call = pl.pallas_call(
    _head_body,
    out_shape=jax.ShapeDtypeStruct((NB, 1), jnp.float32),
)


def kernel(x_s, x_t, edge_attr, edge_index, x_s_batch, x_t_batch, y, params):
    f32 = jnp.float32
    src = edge_index[0].astype(jnp.int32)
    dst = edge_index[1].astype(jnp.int32)
    src3 = src.reshape(NSUB, NCHUNK, K)
    dst3 = dst.reshape(NSUB, NCHUNK, K)
    gidx = jnp.concatenate([src3 + NSN, dst3], axis=0).reshape(-1, K)
    sidx = jnp.concatenate([dst3, src3], axis=0).reshape(-1, K)
    idx2 = jnp.stack([gidx, sidx], axis=1)
    zg = jnp.zeros((RPT, HID), f32)
    zea = jnp.zeros((RPT, DEA), f32)
    bids = jnp.stack([x_t_batch, x_s_batch]).astype(jnp.int32).reshape(2, NSN, 1)

    h2 = jnp.stack([x_t, x_s])
    pools = []
    ea3 = cnt3 = None
    for l in range(NLAYER):
        p = params['layer%d' % l]
        hcat = h2.reshape(TOT, HID)
        if l == 0:
            g, eao, cnto = _spmm_ea_call(hcat, idx2, edge_attr, zg, zea)
            ea3 = eao.reshape(2, NPAD, DEA)
            cnt3 = cnto.reshape(2, NPAD, DEA)
        else:
            g = _spmm_call(hcat, idx2, zg)
        h2, pool = _layer_call(
            h2, g.reshape(2, NPAD, HID), ea3, cnt3,
            bids,
            jnp.stack([p['Ws2t'], p['Wt2s']]),
            jnp.stack([p['We2t'], p['We2s']]),
            jnp.stack([p['Wt_self'], p['Ws_self']]),
            jnp.stack([p['bt'], p['bs']]).reshape(2, 1, HID))
        pools.append(pool)

    return _head_call(
        pools[0], pools[1], pools[2],
        y, params['W_cfg'], params['b_cfg'].reshape(1, 16),
        params['W_pred'], params['b_pred'].reshape(1, 1))


# speedup vs baseline: 6.0056x; 1.0129x over previous
import jax
import jax.numpy as jnp
from jax import lax
from jax.experimental import pallas as pl
from jax.experimental.pallas import tpu as pltpu
from jax.experimental.pallas import tpu_sc as plsc

NSN = 10000
NTN = 10000
TOT = NSN + NTN
HID = 128
EDG = 320000
NB = 64
NLAYER = 3
DEA = 16
EAW = 32
CNT = 16

NC = 2
NSUB = 16
K = 80
NCHUNK = EDG // (NSUB * K)
RPT = 640
NPAD = NSUB * RPT

_sc_mesh = plsc.VectorSubcoreMesh(
    core_axis_name="c", subcore_axis_name="s", num_cores=NC, num_subcores=NSUB)


def _spmm_loop(hcat_hbm, idx_hbm, acc, ibs, isems, rb0, rb1,
               gsem0, gsem1, ssem0, ssem1, base, extra0=None, extra1=None):
    ib00, ib01, ib10, ib11 = ibs
    is00, is01, is10, is11 = isems

    def fetch(j, ib, isem):
        pltpu.async_copy(idx_hbm.at[base + j], ib, isem)

    def wfetch(ib, isem):
        pltpu.make_async_copy(idx_hbm.at[base], ib, isem).wait()

    def gath(ib, rb, gsem):
        pltpu.async_copy(hcat_hbm.at[ib.at[0]], rb, gsem)

    def wgath(ib, rb, gsem):
        pltpu.make_async_copy(hcat_hbm.at[ib.at[0]], rb, gsem).wait()

    def scat(ib, rb, ssem):
        pltpu.async_copy(rb, acc.at[ib.at[1]], ssem, add=True)

    def wscat(ib, rb, ssem):
        pltpu.make_async_copy(rb, acc.at[ib.at[1]], ssem).wait()

    def quad(j0, tail):
        wgath(ib00, rb0, gsem0)
        scat(ib00, rb0, ssem0)
        if extra0 is not None:
            extra0[0](j0, ib00)
        wgath(ib10, rb1, gsem1)
        scat(ib10, rb1, ssem1)
        if extra1 is not None:
            extra1[0](j0 + 1, ib10)
        wscat(ib00, rb0, ssem0)
        if extra0 is not None:
            extra0[1](ib00)
        fetch(j0 + 4, ib00, is00)
        wfetch(ib01, is01)
        gath(ib01, rb0, gsem0)
        wscat(ib10, rb1, ssem1)
        if extra1 is not None:
            extra1[1](ib10)
        fetch(j0 + 5, ib10, is10)
        wfetch(ib11, is11)
        gath(ib11, rb1, gsem1)

        wgath(ib01, rb0, gsem0)
        scat(ib01, rb0, ssem0)
        if extra0 is not None:
            extra0[0](j0 + 2, ib01)
        wgath(ib11, rb1, gsem1)
        scat(ib11, rb1, ssem1)
        if extra1 is not None:
            extra1[0](j0 + 3, ib11)
        wscat(ib01, rb0, ssem0)
        if extra0 is not None:
            extra0[1](ib01)
        if not tail:
            fetch(j0 + 6, ib01, is01)
        wfetch(ib00, is00)
        gath(ib00, rb0, gsem0)
        wscat(ib11, rb1, ssem1)
        if extra1 is not None:
            extra1[1](ib11)
        if not tail:
            fetch(j0 + 7, ib11, is11)
        wfetch(ib10, is10)
        gath(ib10, rb1, gsem1)

    fetch(0, ib00, is00)
    fetch(1, ib10, is10)
    fetch(2, ib01, is01)
    fetch(3, ib11, is11)
    wfetch(ib00, is00)
    gath(ib00, rb0, gsem0)
    wfetch(ib10, is10)
    gath(ib10, rb1, gsem1)

    def step(g, carry):
        quad(g * 4, False)
        return carry

    lax.fori_loop(0, (NCHUNK - 6) // 4, step, 0)
    j0 = NCHUNK - 6
    quad(j0, True)
    wgath(ib00, rb0, gsem0)
    scat(ib00, rb0, ssem0)
    if extra0 is not None:
        extra0[0](j0 + 4, ib00)
    wgath(ib10, rb1, gsem1)
    scat(ib10, rb1, ssem1)
    if extra1 is not None:
        extra1[0](j0 + 5, ib10)
    wscat(ib00, rb0, ssem0)
    if extra0 is not None:
        extra0[1](ib00)
    wscat(ib10, rb1, ssem1)
    if extra1 is not None:
        extra1[1](ib10)


def _spmm_body(hcat_hbm, idx_hbm, zg_hbm, out_hbm,
               acc, ib00, ib01, ib10, ib11, rb0, rb1,
               is00, is01, is10, is11, gsem0, gsem1, ssem0, ssem1):
    c = lax.axis_index("c")
    s = lax.axis_index("s")
    pltpu.sync_copy(zg_hbm, acc.at[pl.ds(s * RPT, RPT)])
    plsc.subcore_barrier()
    _spmm_loop(hcat_hbm, idx_hbm, acc, (ib00, ib01, ib10, ib11),
               (is00, is01, is10, is11), rb0, rb1,
               gsem0, gsem1, ssem0, ssem1, (c * NSUB + s) * NCHUNK)
    plsc.subcore_barrier()
    pltpu.sync_copy(acc.at[pl.ds(s * RPT, RPT)],
                    out_hbm.at[pl.ds(c * NPAD + s * RPT, RPT)])


_spmm_call = pl.kernel(
    _spmm_body,
    out_type=jax.ShapeDtypeStruct((2 * NPAD, HID), jnp.float32),
    mesh=_sc_mesh,
    scratch_types=[
        pltpu.VMEM_SHARED((NPAD, HID), jnp.float32),
        pltpu.VMEM((2, K), jnp.int32),
        pltpu.VMEM((2, K), jnp.int32),
        pltpu.VMEM((2, K), jnp.int32),
        pltpu.VMEM((2, K), jnp.int32),
        pltpu.VMEM((K, HID), jnp.float32),
        pltpu.VMEM((K, HID), jnp.float32),
        pltpu.SemaphoreType.DMA,
        pltpu.SemaphoreType.DMA,
        pltpu.SemaphoreType.DMA,
        pltpu.SemaphoreType.DMA,
        pltpu.SemaphoreType.DMA,
        pltpu.SemaphoreType.DMA,
        pltpu.SemaphoreType.DMA,
        pltpu.SemaphoreType.DMA,
    ],
)


def _spmm_ea_body(hcat_hbm, idx_hbm, ea_hbm, zg_hbm, zea_hbm,
                  out_hbm, oea_hbm, ocnt_hbm,
                  acc, eacc, cacc, ib00, ib01, ib10, ib11, rb0, rb1,
                  eb0, eb1, ones, is00, is01, is10, is11,
                  gsem0, gsem1, ssem0, ssem1, esem0, esem1, csem0, csem1):
    c = lax.axis_index("c")
    s = lax.axis_index("s")
    pltpu.sync_copy(zg_hbm, acc.at[pl.ds(s * RPT, RPT)])
    pltpu.sync_copy(zea_hbm, eacc.at[pl.ds(s * RPT, RPT)])
    pltpu.sync_copy(zea_hbm, cacc.at[pl.ds(s * RPT, RPT)])
    one_row = jnp.zeros((16,), jnp.float32) + 1.0

    def fill(r, carry):
        ones[r, pl.ds(0, 16)] = one_row
        return carry

    lax.fori_loop(0, K, fill, 0)
    plsc.subcore_barrier()
    ebase = s * (NCHUNK * K)

    def issue(j, ib, eb, esem, csem):
        pltpu.sync_copy(ea_hbm.at[pl.ds(ebase + j * K, K)], eb)
        pltpu.async_copy(eb, eacc.at[ib.at[1]], esem, add=True)
        pltpu.async_copy(ones, cacc.at[ib.at[1]], csem, add=True)

    def drain(ib, eb, esem, csem):
        pltpu.make_async_copy(eb, eacc.at[ib.at[1]], esem).wait()
        pltpu.make_async_copy(ones, cacc.at[ib.at[1]], csem).wait()

    _spmm_loop(hcat_hbm, idx_hbm, acc, (ib00, ib01, ib10, ib11),
               (is00, is01, is10, is11), rb0, rb1,
               gsem0, gsem1, ssem0, ssem1, (c * NSUB + s) * NCHUNK,
               extra0=(lambda j, ib: issue(j, ib, eb0, esem0, csem0),
                       lambda ib: drain(ib, eb0, esem0, csem0)),
               extra1=(lambda j, ib: issue(j, ib, eb1, esem1, csem1),
                       lambda ib: drain(ib, eb1, esem1, csem1)))
    plsc.subcore_barrier()
    pltpu.sync_copy(acc.at[pl.ds(s * RPT, RPT)],
                    out_hbm.at[pl.ds(c * NPAD + s * RPT, RPT)])
    pltpu.sync_copy(eacc.at[pl.ds(s * RPT, RPT)],
                    oea_hbm.at[pl.ds(c * NPAD + s * RPT, RPT)])
    pltpu.sync_copy(cacc.at[pl.ds(s * RPT, RPT)],
                    ocnt_hbm.at[pl.ds(c * NPAD + s * RPT, RPT)])


_spmm_ea_call = pl.kernel(
    _spmm_ea_body,
    out_type=(jax.ShapeDtypeStruct((2 * NPAD, HID), jnp.float32),
              jax.ShapeDtypeStruct((2 * NPAD, DEA), jnp.float32),
              jax.ShapeDtypeStruct((2 * NPAD, DEA), jnp.float32)),
    mesh=_sc_mesh,
    compiler_params=pltpu.CompilerParams(use_tc_tiling_on_sc=False),
    scratch_types=[
        pltpu.VMEM_SHARED((NPAD, HID), jnp.float32),
        pltpu.VMEM_SHARED((NPAD, DEA), jnp.float32),
        pltpu.VMEM_SHARED((NPAD, DEA), jnp.float32),
        pltpu.VMEM((2, K), jnp.int32),
        pltpu.VMEM((2, K), jnp.int32),
        pltpu.VMEM((2, K), jnp.int32),
        pltpu.VMEM((2, K), jnp.int32),
        pltpu.VMEM((K, HID), jnp.float32),
        pltpu.VMEM((K, HID), jnp.float32),
        pltpu.VMEM((K, DEA), jnp.float32),
        pltpu.VMEM((K, DEA), jnp.float32),
        pltpu.VMEM((K, DEA), jnp.float32),
        pltpu.SemaphoreType.DMA,
        pltpu.SemaphoreType.DMA,
        pltpu.SemaphoreType.DMA,
        pltpu.SemaphoreType.DMA,
        pltpu.SemaphoreType.DMA,
        pltpu.SemaphoreType.DMA,
        pltpu.SemaphoreType.DMA,
        pltpu.SemaphoreType.DMA,
        pltpu.SemaphoreType.DMA,
        pltpu.SemaphoreType.DMA,
        pltpu.SemaphoreType.DMA,
        pltpu.SemaphoreType.DMA,
    ],
)


RBLK = 2000
GRID = NSN // RBLK


def _layer_body(h2, g2, ea2, cnt2, bids, wx, we, wself, bias,
                hout_ref, pool_ref):
    f32 = jnp.float32
    i = pl.program_id(1)
    iot = lax.broadcasted_iota(jnp.int32, (1, NB), 1)

    @pl.when(i == 0)
    def _():
        pool_ref[...] = jnp.zeros(pool_ref.shape, f32)

    inv = 1.0 / jnp.maximum(cnt2[0][:, 0:1], 1.0)
    agg = (jnp.dot(g2[0], wx[0], preferred_element_type=f32)
           + jnp.dot(ea2[0], we[0], preferred_element_type=f32)) * inv
    nh = jnp.maximum(
        jnp.dot(h2[0], wself[0], preferred_element_type=f32)
        + agg + bias[0], 0.0)
    hout_ref[0] = nh
    mask = jnp.where(bids[0] == iot, 1.0, 0.0)
    pool_ref[0] += lax.dot_general(
        mask, nh, (((0,), (0,)), ((), ())), preferred_element_type=f32)


_layer_call = pl.pallas_call(
    _layer_body,
    grid=(2, GRID),
    in_specs=[
        pl.BlockSpec((1, RBLK, HID), lambda b, i: (b, i, 0)),
        pl.BlockSpec((1, RBLK, HID), lambda b, i: (b, i, 0)),
        pl.BlockSpec((1, RBLK, DEA), lambda b, i: (b, i, 0)),
        pl.BlockSpec((1, RBLK, DEA), lambda b, i: (b, i, 0)),
        pl.BlockSpec((1, RBLK, 1), lambda b, i: (b, i, 0)),
        pl.BlockSpec((1, HID, HID), lambda b, i: (b, 0, 0)),
        pl.BlockSpec((1, DEA, HID), lambda b, i: (b, 0, 0)),
        pl.BlockSpec((1, HID, HID), lambda b, i: (b, 0, 0)),
        pl.BlockSpec((1, 1, HID), lambda b, i: (b, 0, 0)),
    ],
    out_specs=[
        pl.BlockSpec((1, RBLK, HID), lambda b, i: (b, i, 0)),
        pl.BlockSpec((1, NB, HID), lambda b, i: (b, 0, 0)),
    ],
    out_shape=[
        jax.ShapeDtypeStruct((2, NSN, HID), jnp.float32),
        jax.ShapeDtypeStruct((2, NB, HID), jnp.float32),
    ],
)


def _head_body(p0, p1, p2, y, wcfg, bcfg, wp, bp, out_ref):
    f32 = jnp.float32
    yemb = jnp.dot(y[...], wcfg[...], preferred_element_type=f32) + bcfg[...]
    acc = jnp.dot(p0[1], wp[0:128, :], preferred_element_type=f32)
    acc += jnp.dot(p1[1], wp[128:256, :], preferred_element_type=f32)
    acc += jnp.dot(p2[1], wp[256:384, :], preferred_element_type=f32)
    acc += jnp.dot(p0[0], wp[384:512, :], preferred_element_type=f32)
    acc += jnp.dot(p1[0], wp[512:640, :], preferred_element_type=f32)
    acc += jnp.dot(p2[0], wp[640:768, :], preferred_element_type=f32)
    acc += jnp.dot(yemb, wp[768:784, :], preferred_element_type=f32)
    out_ref[...] = acc + bp[...]


_head_call = pl.pallas_call(
    _head_body,
    out_shape=jax.ShapeDtypeStruct((NB, 1), jnp.float32),
)


def kernel(x_s, x_t, edge_attr, edge_index, x_s_batch, x_t_batch, y, params):
    f32 = jnp.float32
    src = edge_index[0].astype(jnp.int32)
    dst = edge_index[1].astype(jnp.int32)
    src3 = src.reshape(NSUB, NCHUNK, K)
    dst3 = dst.reshape(NSUB, NCHUNK, K)
    gidx = jnp.concatenate([src3 + NSN, dst3], axis=0).reshape(-1, K)
    sidx = jnp.concatenate([dst3, src3], axis=0).reshape(-1, K)
    idx2 = jnp.stack([gidx, sidx], axis=1)
    zg = jnp.zeros((RPT, HID), f32)
    zea = jnp.zeros((RPT, DEA), f32)
    bids = jnp.stack([x_t_batch, x_s_batch]).astype(jnp.int32).reshape(2, NSN, 1)

    h2 = jnp.stack([x_t, x_s])
    pools = []
    ea3 = cnt3 = None
    for l in range(NLAYER):
        p = params['layer%d' % l]
        hcat = h2.reshape(TOT, HID)
        if l == 0:
            g, eao, cnto = _spmm_ea_call(hcat, idx2, edge_attr, zg, zea)
            ea3 = eao.reshape(2, NPAD, DEA)
            cnt3 = cnto.reshape(2, NPAD, DEA)
        else:
            g = _spmm_call(hcat, idx2, zg)
        h2, pool = _layer_call(
            h2, g.reshape(2, NPAD, HID), ea3, cnt3,
            bids,
            jnp.stack([p['Ws2t'], p['Wt2s']]),
            jnp.stack([p['We2t'], p['We2s']]),
            jnp.stack([p['Wt_self'], p['Ws_self']]),
            jnp.stack([p['bt'], p['bs']]).reshape(2, 1, HID))
        pools.append(pool)

    return _head_call(
        pools[0], pools[1], pools[2],
        y, params['W_cfg'], params['b_cfg'].reshape(1, 16),
        params['W_pred'], params['b_pred'].reshape(1, 1))


# speedup vs baseline: 6.6135x; 1.1012x over previous
import jax
import jax.numpy as jnp
from jax import lax
from jax.experimental import pallas as pl
from jax.experimental.pallas import tpu as pltpu
from jax.experimental.pallas import tpu_sc as plsc

NSN = 10000
NTN = 10000
TOT = NSN + NTN
HID = 128
EDG = 320000
NB = 64
NLAYER = 3
DEA = 16

NC = 2
NSUB = 16
K = 128
NROW = EDG // K
NCH = NROW // NSUB
NXTRA = NROW - NCH * NSUB
RPT = 640
NPAD = NSUB * RPT

_sc_mesh = plsc.VectorSubcoreMesh(
    core_axis_name="c", subcore_axis_name="s", num_cores=NC, num_subcores=NSUB)


def _spmm_body(hcat_hbm, gidx_hbm, sidx_hbm, zg_hbm, out_hbm,
               acc, ib00, ib01, ib10, ib11, rb0, rb1,
               is00, is01, is10, is11, gsem0, gsem1, ssem0, ssem1):
    c = lax.axis_index("c")
    s = lax.axis_index("s")
    pltpu.sync_copy(zg_hbm, acc.at[pl.ds(s * RPT, RPT)])
    plsc.subcore_barrier()
    base = c * NROW + s * NCH

    def fetch(j, ib, isem):
        pltpu.async_copy(gidx_hbm.at[base + j], ib.at[0], isem)
        pltpu.async_copy(sidx_hbm.at[base + j], ib.at[1], isem)

    def wfetch(ib, isem):
        pltpu.make_async_copy(gidx_hbm.at[pl.ds(0, 2)], ib, isem).wait()

    def gath(ib, rb, gsem):
        pltpu.async_copy(hcat_hbm.at[ib.at[0]], rb, gsem)

    def wgath(ib, rb, gsem):
        pltpu.make_async_copy(hcat_hbm.at[ib.at[0]], rb, gsem).wait()

    def scat(ib, rb, ssem):
        pltpu.async_copy(rb, acc.at[ib.at[1]], ssem, add=True)

    def wscat(ib, rb, ssem):
        pltpu.make_async_copy(rb, acc.at[ib.at[1]], ssem).wait()

    fetch(0, ib00, is00)
    fetch(1, ib10, is10)
    fetch(2, ib01, is01)
    fetch(3, ib11, is11)
    wfetch(ib00, is00)
    gath(ib00, rb0, gsem0)
    wfetch(ib10, is10)
    gath(ib10, rb1, gsem1)

    def quad(g, carry):
        j0 = g * 4
        wgath(ib00, rb0, gsem0)
        scat(ib00, rb0, ssem0)
        wgath(ib10, rb1, gsem1)
        scat(ib10, rb1, ssem1)
        wscat(ib00, rb0, ssem0)
        fetch(j0 + 4, ib00, is00)
        wfetch(ib01, is01)
        gath(ib01, rb0, gsem0)
        wscat(ib10, rb1, ssem1)
        fetch(j0 + 5, ib10, is10)
        wfetch(ib11, is11)
        gath(ib11, rb1, gsem1)

        wgath(ib01, rb0, gsem0)
        scat(ib01, rb0, ssem0)
        wgath(ib11, rb1, gsem1)
        scat(ib11, rb1, ssem1)
        wscat(ib01, rb0, ssem0)
        fetch(j0 + 6, ib01, is01)
        wfetch(ib00, is00)
        gath(ib00, rb0, gsem0)
        wscat(ib11, rb1, ssem1)
        fetch(j0 + 7, ib11, is11)
        wfetch(ib10, is10)
        gath(ib10, rb1, gsem1)
        return carry

    lax.fori_loop(0, (NCH - 4) // 4, quad, 0)
    wgath(ib00, rb0, gsem0)
    scat(ib00, rb0, ssem0)
    wgath(ib10, rb1, gsem1)
    scat(ib10, rb1, ssem1)
    wscat(ib00, rb0, ssem0)
    wfetch(ib01, is01)
    gath(ib01, rb0, gsem0)
    wscat(ib10, rb1, ssem1)
    wfetch(ib11, is11)
    gath(ib11, rb1, gsem1)
    wgath(ib01, rb0, gsem0)
    scat(ib01, rb0, ssem0)
    wgath(ib11, rb1, gsem1)
    scat(ib11, rb1, ssem1)
    wscat(ib01, rb0, ssem0)
    wscat(ib11, rb1, ssem1)

    @pl.when(s < NXTRA)
    def _():
        jx = (NCH * NSUB - s * NCH) + s
        fetch(jx, ib00, is00)
        wfetch(ib00, is00)
        gath(ib00, rb0, gsem0)
        wgath(ib00, rb0, gsem0)
        scat(ib00, rb0, ssem0)
        wscat(ib00, rb0, ssem0)

    plsc.subcore_barrier()
    pltpu.sync_copy(acc.at[pl.ds(s * RPT, RPT)],
                    out_hbm.at[pl.ds(c * NPAD + s * RPT, RPT)])


_spmm_call = pl.kernel(
    _spmm_body,
    out_type=jax.ShapeDtypeStruct((2 * NPAD, HID), jnp.float32),
    mesh=_sc_mesh,
    scratch_types=[
        pltpu.VMEM_SHARED((NPAD, HID), jnp.float32),
        pltpu.VMEM((2, K), jnp.int32),
        pltpu.VMEM((2, K), jnp.int32),
        pltpu.VMEM((2, K), jnp.int32),
        pltpu.VMEM((2, K), jnp.int32),
        pltpu.VMEM((K, HID), jnp.float32),
        pltpu.VMEM((K, HID), jnp.float32),
        pltpu.SemaphoreType.DMA,
        pltpu.SemaphoreType.DMA,
        pltpu.SemaphoreType.DMA,
        pltpu.SemaphoreType.DMA,
        pltpu.SemaphoreType.DMA,
        pltpu.SemaphoreType.DMA,
        pltpu.SemaphoreType.DMA,
        pltpu.SemaphoreType.DMA,
    ],
)


def _ea_body(ea_hbm, sidx_hbm, zea_hbm, oea_hbm, ocnt_hbm,
             eacc, cacc, ib0, ib1, eb0, eb1, ones,
             is0, is1, vs0, vs1, esem0, esem1, csem0, csem1):
    c = lax.axis_index("c")
    s = lax.axis_index("s")
    pltpu.sync_copy(zea_hbm, eacc.at[pl.ds(s * RPT, RPT)])
    pltpu.sync_copy(zea_hbm, cacc.at[pl.ds(s * RPT, RPT)])
    one_row = jnp.zeros((16,), jnp.float32) + 1.0

    def fill(r, carry):
        ones[r, pl.ds(0, 16)] = one_row
        return carry

    lax.fori_loop(0, K, fill, 0)
    plsc.subcore_barrier()
    base = c * NROW + s * NCH
    vbase = s * (NCH * K)

    def fetch(j, ib, isem):
        pltpu.async_copy(sidx_hbm.at[base + j], ib, isem)

    def wfetch(ib, isem):
        pltpu.make_async_copy(sidx_hbm.at[base], ib, isem).wait()

    def vload(j, eb, vsem):
        pltpu.async_copy(ea_hbm.at[pl.ds(vbase + j * K, K)], eb, vsem)

    def wvload(eb, vsem):
        pltpu.make_async_copy(ea_hbm.at[pl.ds(0, K)], eb, vsem).wait()

    def scat(ib, eb, esem, csem):
        pltpu.async_copy(eb, eacc.at[ib], esem, add=True)
        pltpu.async_copy(ones, cacc.at[ib], csem, add=True)

    def wscat(ib, eb, esem, csem):
        pltpu.make_async_copy(eb, eacc.at[ib], esem).wait()
        pltpu.make_async_copy(ones, cacc.at[ib], csem).wait()

    fetch(0, ib0, is0)
    fetch(1, ib1, is1)
    vload(0, eb0, vs0)
    vload(1, eb1, vs1)

    def pair(g, carry):
        j0 = g * 2
        wfetch(ib0, is0)
        wvload(eb0, vs0)
        scat(ib0, eb0, esem0, csem0)
        wfetch(ib1, is1)
        wvload(eb1, vs1)
        scat(ib1, eb1, esem1, csem1)
        wscat(ib0, eb0, esem0, csem0)
        fetch(j0 + 2, ib0, is0)
        vload(j0 + 2, eb0, vs0)
        wscat(ib1, eb1, esem1, csem1)
        fetch(j0 + 3, ib1, is1)
        vload(j0 + 3, eb1, vs1)
        return carry

    lax.fori_loop(0, (NCH - 2) // 2, pair, 0)
    wfetch(ib0, is0)
    wvload(eb0, vs0)
    scat(ib0, eb0, esem0, csem0)
    wfetch(ib1, is1)
    wvload(eb1, vs1)
    scat(ib1, eb1, esem1, csem1)
    wscat(ib0, eb0, esem0, csem0)
    wscat(ib1, eb1, esem1, csem1)

    @pl.when(s < NXTRA)
    def _():
        jx = (NCH * NSUB - s * NCH) + s
        fetch(jx, ib0, is0)
        vload((NCH * NSUB + s) - s * NCH, eb0, vs0)
        wfetch(ib0, is0)
        wvload(eb0, vs0)
        scat(ib0, eb0, esem0, csem0)
        wscat(ib0, eb0, esem0, csem0)

    plsc.subcore_barrier()
    pltpu.sync_copy(eacc.at[pl.ds(s * RPT, RPT)],
                    oea_hbm.at[pl.ds(c * NPAD + s * RPT, RPT)])
    pltpu.sync_copy(cacc.at[pl.ds(s * RPT, RPT)],
                    ocnt_hbm.at[pl.ds(c * NPAD + s * RPT, RPT)])


_ea_call = pl.kernel(
    _ea_body,
    out_type=(jax.ShapeDtypeStruct((2 * NPAD, DEA), jnp.float32),
              jax.ShapeDtypeStruct((2 * NPAD, DEA), jnp.float32)),
    mesh=_sc_mesh,
    compiler_params=pltpu.CompilerParams(use_tc_tiling_on_sc=False),
    scratch_types=[
        pltpu.VMEM_SHARED((NPAD, DEA), jnp.float32),
        pltpu.VMEM_SHARED((NPAD, DEA), jnp.float32),
        pltpu.VMEM((K,), jnp.int32),
        pltpu.VMEM((K,), jnp.int32),
        pltpu.VMEM((K, DEA), jnp.float32),
        pltpu.VMEM((K, DEA), jnp.float32),
        pltpu.VMEM((K, DEA), jnp.float32),
        pltpu.SemaphoreType.DMA,
        pltpu.SemaphoreType.DMA,
        pltpu.SemaphoreType.DMA,
        pltpu.SemaphoreType.DMA,
        pltpu.SemaphoreType.DMA,
        pltpu.SemaphoreType.DMA,
        pltpu.SemaphoreType.DMA,
        pltpu.SemaphoreType.DMA,
    ],
)


RBLK = 2000
GRID = NSN // RBLK


def _layer_body(h2, g2, ea2, cnt2, bids, wx, we, wself, bias,
                hout_ref, pool_ref):
    f32 = jnp.float32
    i = pl.program_id(1)
    iot = lax.broadcasted_iota(jnp.int32, (1, NB), 1)

    @pl.when(i == 0)
    def _():
        pool_ref[...] = jnp.zeros(pool_ref.shape, f32)

    inv = 1.0 / jnp.maximum(cnt2[0][:, 0:1], 1.0)
    agg = (jnp.dot(g2[0], wx[0], preferred_element_type=f32)
           + jnp.dot(ea2[0], we[0], preferred_element_type=f32)) * inv
    nh = jnp.maximum(
        jnp.dot(h2[0], wself[0], preferred_element_type=f32)
        + agg + bias[0], 0.0)
    hout_ref[0] = nh
    mask = jnp.where(bids[0] == iot, 1.0, 0.0)
    pool_ref[0] += lax.dot_general(
        mask, nh, (((0,), (0,)), ((), ())), preferred_element_type=f32)


_layer_call = pl.pallas_call(
    _layer_body,
    grid=(2, GRID),
    in_specs=[
        pl.BlockSpec((1, RBLK, HID), lambda b, i: (b, i, 0)),
        pl.BlockSpec((1, RBLK, HID), lambda b, i: (b, i, 0)),
        pl.BlockSpec((1, RBLK, DEA), lambda b, i: (b, i, 0)),
        pl.BlockSpec((1, RBLK, DEA), lambda b, i: (b, i, 0)),
        pl.BlockSpec((1, RBLK, 1), lambda b, i: (b, i, 0)),
        pl.BlockSpec((1, HID, HID), lambda b, i: (b, 0, 0)),
        pl.BlockSpec((1, DEA, HID), lambda b, i: (b, 0, 0)),
        pl.BlockSpec((1, HID, HID), lambda b, i: (b, 0, 0)),
        pl.BlockSpec((1, 1, HID), lambda b, i: (b, 0, 0)),
    ],
    out_specs=[
        pl.BlockSpec((1, RBLK, HID), lambda b, i: (b, i, 0)),
        pl.BlockSpec((1, NB, HID), lambda b, i: (b, 0, 0)),
    ],
    out_shape=[
        jax.ShapeDtypeStruct((2, NSN, HID), jnp.float32),
        jax.ShapeDtypeStruct((2, NB, HID), jnp.float32),
    ],
)


def _head_body(p0, p1, p2, y, wcfg, bcfg, wp, bp, out_ref):
    f32 = jnp.float32
    yemb = jnp.dot(y[...], wcfg[...], preferred_element_type=f32) + bcfg[...]
    acc = jnp.dot(p0[1], wp[0:128, :], preferred_element_type=f32)
    acc += jnp.dot(p1[1], wp[128:256, :], preferred_element_type=f32)
    acc += jnp.dot(p2[1], wp[256:384, :], preferred_element_type=f32)
    acc += jnp.dot(p0[0], wp[384:512, :], preferred_element_type=f32)
    acc += jnp.dot(p1[0], wp[512:640, :], preferred_element_type=f32)
    acc += jnp.dot(p2[0], wp[640:768, :], preferred_element_type=f32)
    acc += jnp.dot(yemb, wp[768:784, :], preferred_element_type=f32)
    out_ref[...] = acc + bp[...]


_head_call = pl.pallas_call(
    _head_body,
    out_shape=jax.ShapeDtypeStruct((NB, 1), jnp.float32),
)


def kernel(x_s, x_t, edge_attr, edge_index, x_s_batch, x_t_batch, y, params):
    f32 = jnp.float32
    src = edge_index[0].astype(jnp.int32).reshape(NROW, K)
    dst = edge_index[1].astype(jnp.int32).reshape(NROW, K)
    gidx = jnp.concatenate([src + NSN, dst], axis=0)
    sidx = jnp.concatenate([dst, src], axis=0)
    zg = jnp.zeros((RPT, HID), f32)
    zea = jnp.zeros((RPT, DEA), f32)
    bids = jnp.stack([x_t_batch, x_s_batch]).astype(jnp.int32).reshape(2, NSN, 1)

    eao, cnto = _ea_call(edge_attr, sidx, zea)
    ea3 = eao.reshape(2, NPAD, DEA)
    cnt3 = cnto.reshape(2, NPAD, DEA)

    h2 = jnp.stack([x_t, x_s])
    pools = []
    for l in range(NLAYER):
        p = params['layer%d' % l]
        g = _spmm_call(h2.reshape(TOT, HID), gidx, sidx, zg)
        h2, pool = _layer_call(
            h2, g.reshape(2, NPAD, HID), ea3, cnt3,
            bids,
            jnp.stack([p['Ws2t'], p['Wt2s']]),
            jnp.stack([p['We2t'], p['We2s']]),
            jnp.stack([p['Wt_self'], p['Ws_self']]),
            jnp.stack([p['bt'], p['bs']]).reshape(2, 1, HID))
        pools.append(pool)

    return _head_call(
        pools[0], pools[1], pools[2],
        y, params['W_cfg'], params['b_cfg'].reshape(1, 16),
        params['W_pred'], params['b_pred'].reshape(1, 1))


# speedup vs baseline: 6.6165x; 1.0004x over previous
import jax
import jax.numpy as jnp
from jax import lax
from jax.experimental import pallas as pl
from jax.experimental.pallas import tpu as pltpu
from jax.experimental.pallas import tpu_sc as plsc

NSN = 10000
NTN = 10000
TOT = NSN + NTN
HID = 128
EDG = 320000
NB = 64
NLAYER = 3
DEA = 16

NC = 2
NSUB = 16
K = 128
NROW = EDG // K
NCH = NROW // NSUB
NXTRA = NROW - NCH * NSUB
RPT = 640
NPAD = NSUB * RPT

_sc_mesh = plsc.VectorSubcoreMesh(
    core_axis_name="c", subcore_axis_name="s", num_cores=NC, num_subcores=NSUB)


def _spmm_body(hpair_hbm, eidx_hbm, zg_hbm, out_hbm,
               acc, ib00, ib01, ib10, ib11, rb0, rb1,
               is00, is01, is10, is11, gsem0, gsem1, ssem0, ssem1):
    c = lax.axis_index("c")
    s = lax.axis_index("s")
    pltpu.sync_copy(zg_hbm, acc.at[pl.ds(s * RPT, RPT)])
    plsc.subcore_barrier()
    base = s * NCH
    htab = hpair_hbm.at[1 - c]

    def fetch(j, ib, isem):
        pltpu.async_copy(eidx_hbm.at[c, base + j], ib.at[0], isem)
        pltpu.async_copy(eidx_hbm.at[1 - c, base + j], ib.at[1], isem)

    def wfetch(ib, isem):
        pltpu.make_async_copy(eidx_hbm.at[0, pl.ds(0, 2)], ib, isem).wait()

    def gath(ib, rb, gsem):
        pltpu.async_copy(htab.at[ib.at[0]], rb, gsem)

    def wgath(ib, rb, gsem):
        pltpu.make_async_copy(htab.at[ib.at[0]], rb, gsem).wait()

    def scat(ib, rb, ssem):
        pltpu.async_copy(rb, acc.at[ib.at[1]], ssem, add=True)

    def wscat(ib, rb, ssem):
        pltpu.make_async_copy(rb, acc.at[ib.at[1]], ssem).wait()

    fetch(0, ib00, is00)
    fetch(1, ib10, is10)
    fetch(2, ib01, is01)
    fetch(3, ib11, is11)
    wfetch(ib00, is00)
    gath(ib00, rb0, gsem0)
    wfetch(ib10, is10)
    gath(ib10, rb1, gsem1)

    def quad(g, carry):
        j0 = g * 4
        wgath(ib00, rb0, gsem0)
        scat(ib00, rb0, ssem0)
        wgath(ib10, rb1, gsem1)
        scat(ib10, rb1, ssem1)
        wscat(ib00, rb0, ssem0)
        fetch(j0 + 4, ib00, is00)
        wfetch(ib01, is01)
        gath(ib01, rb0, gsem0)
        wscat(ib10, rb1, ssem1)
        fetch(j0 + 5, ib10, is10)
        wfetch(ib11, is11)
        gath(ib11, rb1, gsem1)

        wgath(ib01, rb0, gsem0)
        scat(ib01, rb0, ssem0)
        wgath(ib11, rb1, gsem1)
        scat(ib11, rb1, ssem1)
        wscat(ib01, rb0, ssem0)
        fetch(j0 + 6, ib01, is01)
        wfetch(ib00, is00)
        gath(ib00, rb0, gsem0)
        wscat(ib11, rb1, ssem1)
        fetch(j0 + 7, ib11, is11)
        wfetch(ib10, is10)
        gath(ib10, rb1, gsem1)
        return carry

    lax.fori_loop(0, (NCH - 4) // 4, quad, 0)
    wgath(ib00, rb0, gsem0)
    scat(ib00, rb0, ssem0)
    wgath(ib10, rb1, gsem1)
    scat(ib10, rb1, ssem1)
    wscat(ib00, rb0, ssem0)
    wfetch(ib01, is01)
    gath(ib01, rb0, gsem0)
    wscat(ib10, rb1, ssem1)
    wfetch(ib11, is11)
    gath(ib11, rb1, gsem1)
    wgath(ib01, rb0, gsem0)
    scat(ib01, rb0, ssem0)
    wgath(ib11, rb1, gsem1)
    scat(ib11, rb1, ssem1)
    wscat(ib01, rb0, ssem0)
    wscat(ib11, rb1, ssem1)

    @pl.when(s < NXTRA)
    def _():
        jx = (NCH * NSUB - s * NCH) + s
        fetch(jx, ib00, is00)
        wfetch(ib00, is00)
        gath(ib00, rb0, gsem0)
        wgath(ib00, rb0, gsem0)
        scat(ib00, rb0, ssem0)
        wscat(ib00, rb0, ssem0)

    plsc.subcore_barrier()
    pltpu.sync_copy(acc.at[pl.ds(s * RPT, RPT)],
                    out_hbm.at[pl.ds(c * NPAD + s * RPT, RPT)])


_spmm_call = pl.kernel(
    _spmm_body,
    out_type=jax.ShapeDtypeStruct((2 * NPAD, HID), jnp.float32),
    mesh=_sc_mesh,
    scratch_types=[
        pltpu.VMEM_SHARED((NPAD, HID), jnp.float32),
        pltpu.VMEM((2, K), jnp.int32),
        pltpu.VMEM((2, K), jnp.int32),
        pltpu.VMEM((2, K), jnp.int32),
        pltpu.VMEM((2, K), jnp.int32),
        pltpu.VMEM((K, HID), jnp.float32),
        pltpu.VMEM((K, HID), jnp.float32),
        pltpu.SemaphoreType.DMA,
        pltpu.SemaphoreType.DMA,
        pltpu.SemaphoreType.DMA,
        pltpu.SemaphoreType.DMA,
        pltpu.SemaphoreType.DMA,
        pltpu.SemaphoreType.DMA,
        pltpu.SemaphoreType.DMA,
        pltpu.SemaphoreType.DMA,
    ],
)


def _ea_body(ea_hbm, eidx_hbm, zea_hbm, oea_hbm, ocnt_hbm,
             eacc, cacc, ib0, ib1, eb0, eb1, ones,
             is0, is1, vs0, vs1, esem0, esem1, csem0, csem1):
    c = lax.axis_index("c")
    s = lax.axis_index("s")
    pltpu.sync_copy(zea_hbm, eacc.at[pl.ds(s * RPT, RPT)])
    pltpu.sync_copy(zea_hbm, cacc.at[pl.ds(s * RPT, RPT)])
    one_row = jnp.zeros((16,), jnp.float32) + 1.0

    def fill(r, carry):
        ones[r, pl.ds(0, 16)] = one_row
        return carry

    lax.fori_loop(0, K, fill, 0)
    plsc.subcore_barrier()
    base = s * NCH
    vbase = s * (NCH * K)

    def fetch(j, ib, isem):
        pltpu.async_copy(eidx_hbm.at[1 - c, base + j], ib, isem)

    def wfetch(ib, isem):
        pltpu.make_async_copy(eidx_hbm.at[0, base], ib, isem).wait()

    def vload(j, eb, vsem):
        pltpu.async_copy(ea_hbm.at[pl.ds(vbase + j * K, K)], eb, vsem)

    def wvload(eb, vsem):
        pltpu.make_async_copy(ea_hbm.at[pl.ds(0, K)], eb, vsem).wait()

    def scat(ib, eb, esem, csem):
        pltpu.async_copy(eb, eacc.at[ib], esem, add=True)
        pltpu.async_copy(ones, cacc.at[ib], csem, add=True)

    def wscat(ib, eb, esem, csem):
        pltpu.make_async_copy(eb, eacc.at[ib], esem).wait()
        pltpu.make_async_copy(ones, cacc.at[ib], csem).wait()

    fetch(0, ib0, is0)
    fetch(1, ib1, is1)
    vload(0, eb0, vs0)
    vload(1, eb1, vs1)

    def pair(g, carry):
        j0 = g * 2
        wfetch(ib0, is0)
        wvload(eb0, vs0)
        scat(ib0, eb0, esem0, csem0)
        wfetch(ib1, is1)
        wvload(eb1, vs1)
        scat(ib1, eb1, esem1, csem1)
        wscat(ib0, eb0, esem0, csem0)
        fetch(j0 + 2, ib0, is0)
        vload(j0 + 2, eb0, vs0)
        wscat(ib1, eb1, esem1, csem1)
        fetch(j0 + 3, ib1, is1)
        vload(j0 + 3, eb1, vs1)
        return carry

    lax.fori_loop(0, (NCH - 2) // 2, pair, 0)
    wfetch(ib0, is0)
    wvload(eb0, vs0)
    scat(ib0, eb0, esem0, csem0)
    wfetch(ib1, is1)
    wvload(eb1, vs1)
    scat(ib1, eb1, esem1, csem1)
    wscat(ib0, eb0, esem0, csem0)
    wscat(ib1, eb1, esem1, csem1)

    @pl.when(s < NXTRA)
    def _():
        jx = (NCH * NSUB - s * NCH) + s
        fetch(jx, ib0, is0)
        vload((NCH * NSUB + s) - s * NCH, eb0, vs0)
        wfetch(ib0, is0)
        wvload(eb0, vs0)
        scat(ib0, eb0, esem0, csem0)
        wscat(ib0, eb0, esem0, csem0)

    plsc.subcore_barrier()
    pltpu.sync_copy(eacc.at[pl.ds(s * RPT, RPT)],
                    oea_hbm.at[pl.ds(c * NPAD + s * RPT, RPT)])
    pltpu.sync_copy(cacc.at[pl.ds(s * RPT, RPT)],
                    ocnt_hbm.at[pl.ds(c * NPAD + s * RPT, RPT)])


_ea_call = pl.kernel(
    _ea_body,
    out_type=(jax.ShapeDtypeStruct((2 * NPAD, DEA), jnp.float32),
              jax.ShapeDtypeStruct((2 * NPAD, DEA), jnp.float32)),
    mesh=_sc_mesh,
    compiler_params=pltpu.CompilerParams(use_tc_tiling_on_sc=False),
    scratch_types=[
        pltpu.VMEM_SHARED((NPAD, DEA), jnp.float32),
        pltpu.VMEM_SHARED((NPAD, DEA), jnp.float32),
        pltpu.VMEM((K,), jnp.int32),
        pltpu.VMEM((K,), jnp.int32),
        pltpu.VMEM((K, DEA), jnp.float32),
        pltpu.VMEM((K, DEA), jnp.float32),
        pltpu.VMEM((K, DEA), jnp.float32),
        pltpu.SemaphoreType.DMA,
        pltpu.SemaphoreType.DMA,
        pltpu.SemaphoreType.DMA,
        pltpu.SemaphoreType.DMA,
        pltpu.SemaphoreType.DMA,
        pltpu.SemaphoreType.DMA,
        pltpu.SemaphoreType.DMA,
        pltpu.SemaphoreType.DMA,
    ],
)


RBLK = 2000
GRID = NSN // RBLK


def _layer_body(h2, g2, ea2, cnt2, bids, wx, we, wself, bias,
                hout_ref, pool_ref):
    f32 = jnp.float32
    i = pl.program_id(1)
    iot = lax.broadcasted_iota(jnp.int32, (1, NB), 1)

    @pl.when(i == 0)
    def _():
        pool_ref[...] = jnp.zeros(pool_ref.shape, f32)

    inv = 1.0 / jnp.maximum(cnt2[0][:, 0:1], 1.0)
    agg = (jnp.dot(g2[0], wx[0], preferred_element_type=f32)
           + jnp.dot(ea2[0], we[0], preferred_element_type=f32)) * inv
    nh = jnp.maximum(
        jnp.dot(h2[0], wself[0], preferred_element_type=f32)
        + agg + bias[0], 0.0)
    hout_ref[0] = nh
    mask = jnp.where(bids[0] == iot, 1.0, 0.0)
    pool_ref[0] += lax.dot_general(
        mask, nh, (((0,), (0,)), ((), ())), preferred_element_type=f32)


_layer_call = pl.pallas_call(
    _layer_body,
    grid=(2, GRID),
    in_specs=[
        pl.BlockSpec((1, RBLK, HID), lambda b, i: (b, i, 0)),
        pl.BlockSpec((1, RBLK, HID), lambda b, i: (b, i, 0)),
        pl.BlockSpec((1, RBLK, DEA), lambda b, i: (b, i, 0)),
        pl.BlockSpec((1, RBLK, DEA), lambda b, i: (b, i, 0)),
        pl.BlockSpec((1, RBLK, 1), lambda b, i: (b, i, 0)),
        pl.BlockSpec((1, HID, HID), lambda b, i: (b, 0, 0)),
        pl.BlockSpec((1, DEA, HID), lambda b, i: (b, 0, 0)),
        pl.BlockSpec((1, HID, HID), lambda b, i: (b, 0, 0)),
        pl.BlockSpec((1, 1, HID), lambda b, i: (b, 0, 0)),
    ],
    out_specs=[
        pl.BlockSpec((1, RBLK, HID), lambda b, i: (b, i, 0)),
        pl.BlockSpec((1, NB, HID), lambda b, i: (b, 0, 0)),
    ],
    out_shape=[
        jax.ShapeDtypeStruct((2, NSN, HID), jnp.float32),
        jax.ShapeDtypeStruct((2, NB, HID), jnp.float32),
    ],
)


def _head_body(p0, p1, p2, y, wcfg, bcfg, wp, bp, out_ref):
    f32 = jnp.float32
    yemb = jnp.dot(y[...], wcfg[...], preferred_element_type=f32) + bcfg[...]
    acc = jnp.dot(p0[1], wp[0:128, :], preferred_element_type=f32)
    acc += jnp.dot(p1[1], wp[128:256, :], preferred_element_type=f32)
    acc += jnp.dot(p2[1], wp[256:384, :], preferred_element_type=f32)
    acc += jnp.dot(p0[0], wp[384:512, :], preferred_element_type=f32)
    acc += jnp.dot(p1[0], wp[512:640, :], preferred_element_type=f32)
    acc += jnp.dot(p2[0], wp[640:768, :], preferred_element_type=f32)
    acc += jnp.dot(yemb, wp[768:784, :], preferred_element_type=f32)
    out_ref[...] = acc + bp[...]


_head_call = pl.pallas_call(
    _head_body,
    out_shape=jax.ShapeDtypeStruct((NB, 1), jnp.float32),
)


def kernel(x_s, x_t, edge_attr, edge_index, x_s_batch, x_t_batch, y, params):
    f32 = jnp.float32
    eidx = edge_index.astype(jnp.int32).reshape(2, NROW, K)
    zg = jnp.zeros((RPT, HID), f32)
    zea = jnp.zeros((RPT, DEA), f32)
    bids = jnp.stack([x_t_batch, x_s_batch]).astype(jnp.int32).reshape(2, NSN, 1)

    eao, cnto = _ea_call(edge_attr, eidx, zea)
    ea3 = eao.reshape(2, NPAD, DEA)
    cnt3 = cnto.reshape(2, NPAD, DEA)

    h2 = jnp.stack([x_t, x_s])
    pools = []
    for l in range(NLAYER):
        p = params['layer%d' % l]
        g = _spmm_call(h2, eidx, zg)
        h2, pool = _layer_call(
            h2, g.reshape(2, NPAD, HID), ea3, cnt3,
            bids,
            jnp.stack([p['Ws2t'], p['Wt2s']]),
            jnp.stack([p['We2t'], p['We2s']]),
            jnp.stack([p['Wt_self'], p['Ws_self']]),
            jnp.stack([p['bt'], p['bs']]).reshape(2, 1, HID))
        pools.append(pool)

    return _head_call(
        pools[0], pools[1], pools[2],
        y, params['W_cfg'], params['b_cfg'].reshape(1, 16),
        params['W_pred'], params['b_pred'].reshape(1, 1))
